# baseline (speedup 1.0000x reference)
import functools
import math

import jax
import jax.numpy as jnp
from jax import lax
from jax.experimental import pallas as pl
from jax.experimental.pallas import tpu as pltpu
from jax.experimental.pallas import tpu_sc as plsc

_IT = False
_G = 256
_EPS = 1e-5

_NSC = 2
_NTILE = 16


def _sigmoid(x):
    return 1.0 / (1.0 + jnp.exp(-x))


def _silu(x):
    return x * _sigmoid(x)


def _blk(n, target):
    if n % target == 0:
        return target
    return n


def _prologue_node_kernel(x_ref, w_ref, b_ref, o_ref):
    o_ref[...] = jnp.dot(x_ref[...], w_ref[...],
                         preferred_element_type=jnp.float32) + b_ref[...]


def _prologue_edge_kernel(ea_ref, w1_ref, b1_ref, w2_ref, b2_ref, o_ref, *,
                          efb):
    ea = ea_ref[...]
    d = jnp.sqrt(jnp.sum(ea * ea, axis=1, keepdims=True))
    step = 8.0 / (efb - 1)
    centers = lax.broadcasted_iota(jnp.int32, (1, efb), 1).astype(jnp.float32) * step
    gamma = 1.0 / (step * step)
    diff = d - centers
    rbf = jnp.exp(-gamma * diff * diff)
    z = jnp.dot(rbf, w1_ref[...], preferred_element_type=jnp.float32) + b1_ref[...]
    sp = jnp.maximum(z, 0.0) + jnp.log(1.0 + jnp.exp(-jnp.abs(z)))
    o_ref[...] = jnp.dot(sp, w2_ref[...],
                         preferred_element_type=jnp.float32) + b2_ref[...]


def _pack2(lo, hi):
    lo16 = lax.bitcast_convert_type(lo.astype(jnp.bfloat16),
                                    jnp.uint16).astype(jnp.uint32)
    hi16 = lax.bitcast_convert_type(hi.astype(jnp.bfloat16),
                                    jnp.uint16).astype(jnp.uint32)
    return lax.bitcast_convert_type(lo16 | (hi16 << 16), jnp.int32)


def _unpack(x, hi):
    u = lax.bitcast_convert_type(x, jnp.uint32)
    h16 = (u >> 16) if hi else (u & jnp.uint32(0xFFFF))
    b = lax.bitcast_convert_type(h16.astype(jnp.uint16), jnp.bfloat16)
    return b.astype(jnp.float32)


def _store_tables(q, k, v, qk_ref, vd_ref, kv_ref, c):
    nq = q.shape[1] // c
    hp = nq // 2
    for t in range(hp):
        s0 = slice((2 * t) * c, (2 * t + 1) * c)
        s1 = slice((2 * t + 1) * c, (2 * t + 2) * c)
        qk_ref[:, t, :] = _pack2(q[:, s0], q[:, s1])
        qk_ref[:, hp + t, :] = _pack2(k[:, s0], k[:, s1])
        vd_ref[:, t, :] = _pack2(v[:, s0], v[:, s1])
        kv_ref[:, t, :] = _pack2(k[:, s0], k[:, s1])
        kv_ref[:, hp + t, :] = _pack2(v[:, s0], v[:, s1])


def _proj_kernel(p_ref, wq_ref, bq_ref, wk_ref, bk_ref, wv_ref, bv_ref,
                 qk_ref, vd_ref, kv_ref, *, hc, c):
    nf = p_ref[...]
    q = jnp.dot(nf, wq_ref[...], preferred_element_type=jnp.float32) + bq_ref[...]
    k = jnp.dot(nf, wk_ref[...], preferred_element_type=jnp.float32) + bk_ref[...]
    v = jnp.dot(nf, wv_ref[...], preferred_element_type=jnp.float32) + bv_ref[...]
    _store_tables(q, k, v, qk_ref, vd_ref, kv_ref, c)


def _proj_bn_kernel(p_ref, stats_ref, g_ref, b_ref,
                    wq_ref, bq_ref, wk_ref, bk_ref, wv_ref, bv_ref,
                    qk_ref, vd_ref, kv_ref, *, hc, c, n_nodes):
    p = p_ref[...]
    m = stats_ref[0:1, :] / n_nodes
    var = stats_ref[1:2, :] / n_nodes - m * m
    xb = (p - m) * lax.rsqrt(var + _EPS) * g_ref[...] + b_ref[...]
    nf = _silu(xb)
    q = jnp.dot(nf, wq_ref[...], preferred_element_type=jnp.float32) + bq_ref[...]
    k = jnp.dot(nf, wk_ref[...], preferred_element_type=jnp.float32) + bk_ref[...]
    v = jnp.dot(nf, wv_ref[...], preferred_element_type=jnp.float32) + bv_ref[...]
    _store_tables(q, k, v, qk_ref, vd_ref, kv_ref, c)


def _eproj_kernel(ef_ref, we_ref, o_ref):
    o_ref[...] = jnp.dot(ef_ref[...], we_ref[...],
                         preferred_element_type=jnp.float32)


def _edge_kernel(gqk_ref, gvd_ref, gkv_ref, e_ref, wmu_ref, bmu_ref,
                 wml_ref, bml_ref,
                 lag_ref, lab_ref, lmg_ref, lmb_ref, o_ref, *, h, c):
    scale = 1.0 / math.sqrt(3.0 * c)
    f32 = jnp.float32
    wmu = wmu_ref[...]
    bmu = bmu_ref[...]
    wml = wml_ref[...]
    bml = bml_ref[...]
    hp = h // 2
    for hh in range(h):
        sl = slice(hh * c, (hh + 1) * c)
        t, odd = hh // 2, hh % 2
        q = _unpack(gqk_ref[:, t, :], odd)
        k_i = _unpack(gqk_ref[:, hp + t, :], odd)
        v_i = _unpack(gvd_ref[:, t, :], odd)
        k_j = _unpack(gkv_ref[:, t, :], odd)
        v_j = _unpack(gkv_ref[:, hp + t, :], odd)
        e = e_ref[:, sl]
        a1 = q * k_i * scale
        a2 = q * k_j * scale
        a3 = q * e * scale
        s = (jnp.sum(a1, axis=1, keepdims=True)
             + jnp.sum(a2, axis=1, keepdims=True)
             + jnp.sum(a3, axis=1, keepdims=True))
        ss = (jnp.sum(a1 * a1, axis=1, keepdims=True)
              + jnp.sum(a2 * a2, axis=1, keepdims=True)
              + jnp.sum(a3 * a3, axis=1, keepdims=True))
        m = s / (3.0 * c)
        var = ss / (3.0 * c) - m * m
        r = lax.rsqrt(var + _EPS)
        g1 = _sigmoid((a1 - m) * r * lag_ref[:, 0:c] + lab_ref[:, 0:c])
        g2 = _sigmoid((a2 - m) * r * lag_ref[:, c:2 * c] + lab_ref[:, c:2 * c])
        g3 = _sigmoid((a3 - m) * r * lag_ref[:, 2 * c:3 * c] + lab_ref[:, 2 * c:3 * c])
        m2 = (jnp.dot(v_i, wmu[0:c, :], preferred_element_type=jnp.float32)
              + jnp.dot(v_j, wmu[c:2 * c, :], preferred_element_type=jnp.float32)
              + jnp.dot(e, wmu[2 * c:3 * c, :], preferred_element_type=jnp.float32)
              + bmu)
        m3 = (jnp.dot(m2[:, 0:c] * g1, wml[0:c, :],
                      preferred_element_type=jnp.float32)
              + jnp.dot(m2[:, c:2 * c] * g2, wml[c:2 * c, :],
                        preferred_element_type=jnp.float32)
              + jnp.dot(m2[:, 2 * c:3 * c] * g3, wml[2 * c:3 * c, :],
                        preferred_element_type=jnp.float32)
              + bml)
        mm = jnp.mean(m3, axis=1, keepdims=True)
        mv = jnp.mean(m3 * m3, axis=1, keepdims=True) - mm * mm
        msg = (m3 - mm) * lax.rsqrt(mv + _EPS) * lmg_ref[...] + lmb_ref[...]
        o_ref[hh, :, :] = msg


def _make_gather(sl, ep, b):
    per_tile = ep // (_NSC * _NTILE)
    nb = per_tile // b
    npair = nb // 2
    mesh = plsc.VectorSubcoreMesh(core_axis_name="c", subcore_axis_name="s")

    def body(table_hbm, idx_hbm, out_hbm, idx_v, r0, r1, g0, g1, o0, o1):
        cid = lax.axis_index("c")
        sid = lax.axis_index("s")
        wid = sid * _NSC + cid
        base = wid * per_tile
        pltpu.sync_copy(idx_hbm.at[pl.ds(base, per_tile)], idx_v)
        bufs = (r0, r1)
        gsem = (g0, g1)
        osem = (o0, o1)

        def start_g(j, p):
            pltpu.async_copy(table_hbm.at[idx_v.at[pl.ds(j * b, b)]],
                             bufs[p], gsem[p])

        def wait_g(j, p):
            pltpu.make_async_copy(table_hbm.at[idx_v.at[pl.ds(j * b, b)]],
                                  bufs[p], gsem[p]).wait()

        def start_o(j, p):
            pltpu.async_copy(bufs[p], out_hbm.at[pl.ds(base + j * b, b)],
                             osem[p])

        def wait_o(j, p):
            pltpu.make_async_copy(bufs[p],
                                  out_hbm.at[pl.ds(base + j * b, b)],
                                  osem[p]).wait()

        start_g(0, 0)
        start_g(1, 1)

        def pair(j2, carry):
            j = 2 * j2
            wait_g(j, 0)
            start_o(j, 0)
            wait_g(j + 1, 1)
            start_o(j + 1, 1)
            wait_o(j, 0)
            start_g(j + 2, 0)
            wait_o(j + 1, 1)
            start_g(j + 3, 1)
            return carry

        lax.fori_loop(0, npair - 1, pair, 0)
        j = nb - 2
        wait_g(j, 0)
        start_o(j, 0)
        wait_g(j + 1, 1)
        start_o(j + 1, 1)
        wait_o(j, 0)
        wait_o(j + 1, 1)

    def make(table, idx):
        return pl.kernel(
            body,
            out_type=jax.ShapeDtypeStruct((ep, sl, 128), jnp.int32),
            mesh=mesh,
            scratch_types=[
                pltpu.VMEM((per_tile,), jnp.int32),
                pltpu.VMEM((b, sl, 128), jnp.int32),
                pltpu.VMEM((b, sl, 128), jnp.int32),
                pltpu.SemaphoreType.DMA,
                pltpu.SemaphoreType.DMA,
                pltpu.SemaphoreType.DMA,
                pltpu.SemaphoreType.DMA,
            ],
        )(table, idx)

    return make


def _make_scatter(n, ep, h, c):
    ntab = ((n + _NTILE * 8) // (_NTILE * 8)) * (_NTILE * 8)
    br = 128
    per_tile_b = (ep // br) // _NTILE
    npair = per_tile_b // 2
    tab_slice = ntab // _NTILE
    rows_lo = (n // _NTILE) // 8 * 8
    rows_hi = n - rows_lo * (_NTILE - 1)
    vper = c // 16
    mesh = plsc.VectorSubcoreMesh(core_axis_name="c", subcore_axis_name="s")

    @functools.partial(
        pl.kernel,
        out_type=jax.ShapeDtypeStruct((h, n, c), jnp.float32),
        mesh=mesh,
        scratch_types=[
            pltpu.VMEM((128, c), jnp.float32),
            pltpu.VMEM((br, c), jnp.float32),
            pltpu.VMEM((br, c), jnp.float32),
            pltpu.VMEM((128,), jnp.int32),
            pltpu.VMEM((128,), jnp.int32),
            pltpu.VMEM((128,), jnp.int32),
            pltpu.VMEM((128,), jnp.int32),
            pltpu.VMEM_SHARED((ntab, c), jnp.float32),
            pltpu.SemaphoreType.DMA,
            pltpu.SemaphoreType.DMA,
        ],
    )
    def scat(msg_hbm, dstb_hbm, agg_hbm, zero_v, rows0, rows1,
             ia0, ib0, ia1, ib1, table, s0, s1):
        cid = lax.axis_index("c")
        sid = lax.axis_index("s")
        rows = (rows0, rows1)
        ia = (ia0, ia1)
        ib = (ib0, ib1)
        sem = (s0, s1)

        def zbody(i, carry):
            r = i // vper
            col = (i % vper) * 16
            zero_v[r, pl.ds(col, 16)] = jnp.zeros((16,), jnp.float32)
            return carry

        lax.fori_loop(0, 128 * vper, zbody, 0)
        tab_base = sid * tab_slice

        for ck in range(h // _NSC):
            chunk = cid * (h // _NSC) + ck
            off = 0
            left = tab_slice
            while left > 0:
                step = min(128, left)
                pltpu.sync_copy(zero_v.at[pl.ds(0, step)],
                                table.at[pl.ds(tab_base + off, step)])
                off += step
                left -= step
            plsc.subcore_barrier()

            def load(j, p):
                e0 = (sid * per_tile_b + j) * br
                pltpu.async_copy(dstb_hbm.at[pl.ds(e0, 128)], ia[p], sem[p])
                pltpu.async_copy(msg_hbm.at[chunk, pl.ds(e0, br)], rows[p],
                                 sem[p])

            def wait_load(j, p):
                e0 = (sid * per_tile_b + j) * br
                pltpu.make_async_copy(dstb_hbm.at[pl.ds(e0, 128)], ia[p],
                                      sem[p]).wait()
                pltpu.make_async_copy(msg_hbm.at[chunk, pl.ds(e0, br)],
                                      rows[p], sem[p]).wait()

            def add2(p):
                pltpu.sync_copy(rows[p], table.at[ia[p]], add=True)

            load(0, 0)

            def pair(j2, carry):
                j = 2 * j2
                load(j + 1, 1)
                wait_load(j, 0)
                add2(0)
                load(j + 2, 0)
                wait_load(j + 1, 1)
                add2(1)
                return carry

            lax.fori_loop(0, npair - 1, pair, 0)
            j = per_tile_b - 2
            load(j + 1, 1)
            wait_load(j, 0)
            add2(0)
            wait_load(j + 1, 1)
            add2(1)

            plsc.subcore_barrier()
            out_base = sid * rows_lo

            @pl.when(sid < _NTILE - 1)
            def _copy_lo():
                pltpu.sync_copy(table.at[pl.ds(out_base, rows_lo)],
                                agg_hbm.at[chunk, pl.ds(out_base, rows_lo)])

            @pl.when(sid == _NTILE - 1)
            def _copy_hi():
                base = rows_lo * (_NTILE - 1)
                pltpu.sync_copy(table.at[pl.ds(base, rows_hi)],
                                agg_hbm.at[chunk, pl.ds(base, rows_hi)])

            plsc.subcore_barrier()

    return scat


def _update_kernel(agg_ref, wcat_ref, bcat_ref, o_ref, stats_ref, *, h, c):
    i = pl.program_id(0)
    wcat = wcat_ref[...]
    o = bcat_ref[...]
    for hh in range(h):
        o = o + jnp.dot(agg_ref[hh, :, :], wcat[hh * c:(hh + 1) * c, :],
                        preferred_element_type=jnp.float32)
    o_ref[...] = o

    @pl.when(i == 0)
    def _init():
        stats_ref[...] = jnp.zeros_like(stats_ref)

    s = jnp.sum(o, axis=0, keepdims=True)
    ss = jnp.sum(o * o, axis=0, keepdims=True)
    stats_ref[0:1, :] = stats_ref[0:1, :] + s
    stats_ref[1:2, :] = stats_ref[1:2, :] + ss


def _pool_kernel(p_ref, stats_ref, g_ref, b_ref, batch_ref,
                 wfc_ref, bfc_ref, wout_ref, bout_ref, o_ref, *, n_nodes):
    p = p_ref[...]
    m = stats_ref[0:1, :] / n_nodes
    var = stats_ref[1:2, :] / n_nodes - m * m
    nf = _silu((p - m) * lax.rsqrt(var + _EPS) * g_ref[...] + b_ref[...])
    seg = batch_ref[...]
    gids = lax.broadcasted_iota(jnp.int32, (_G, 1), 0)
    onehot = (seg == gids).astype(jnp.float32)
    pooled = jnp.dot(onehot, nf, preferred_element_type=jnp.float32)
    counts = jnp.sum(onehot, axis=1, keepdims=True)
    pooled = pooled / jnp.maximum(counts, 1.0)
    feat = _silu(jnp.dot(pooled, wfc_ref[...],
                         preferred_element_type=jnp.float32) + bfc_ref[...])
    out = jnp.sum(feat * wout_ref[...], axis=1, keepdims=True) + bout_ref[...]
    o_ref[...] = out


def kernel(x, edge_attr, edge_index, batch, W_atom, b_atom, W_rbf1, b_rbf1,
           W_rbf2, b_rbf2, Wq, bq, Wk, bk, Wv, bv, We, Wcat, bcat, Wmu, bmu,
           Wml, bml, ln_msg_g, ln_msg_b, ln_a_g, ln_a_b, bn_g, bn_b,
           W_fc, b_fc, W_out, b_out):
    n, aif = x.shape
    e = edge_attr.shape[0]
    nfdim = W_atom.shape[1]
    efb = W_rbf1.shape[0]
    l_layers = Wq.shape[0]
    c = Wcat.shape[2]
    h = Wq.shape[2] // c
    hc = h * c

    ep = ((e + 4095) // 4096) * 4096
    pad = ep - e
    src = jnp.concatenate([edge_index[0], jnp.zeros((pad,), jnp.int32)])
    dst = jnp.concatenate([edge_index[1], jnp.zeros((pad,), jnp.int32)])
    dst_b = jnp.concatenate(
        [edge_index[1], jnp.full((pad,), n, jnp.int32)])
    edge_attr = jnp.concatenate(
        [edge_attr, jnp.zeros((pad, edge_attr.shape[1]), edge_attr.dtype)])

    f32 = jnp.float32
    row2 = lambda a: a.reshape(1, -1)

    nf0 = pl.pallas_call(
        _prologue_node_kernel,
        out_shape=jax.ShapeDtypeStruct((n, nfdim), f32),
        interpret=_IT,
    )(x, W_atom, row2(b_atom))

    eblk = _blk(ep, 5120)
    ef = pl.pallas_call(
        functools.partial(_prologue_edge_kernel, efb=efb),
        grid=(ep // eblk,),
        in_specs=[
            pl.BlockSpec((eblk, edge_attr.shape[1]), lambda i: (i, 0)),
            pl.BlockSpec((efb, nfdim), lambda i: (0, 0)),
            pl.BlockSpec((1, nfdim), lambda i: (0, 0)),
            pl.BlockSpec((nfdim, nfdim), lambda i: (0, 0)),
            pl.BlockSpec((1, nfdim), lambda i: (0, 0)),
        ],
        out_specs=pl.BlockSpec((eblk, nfdim), lambda i: (i, 0)),
        out_shape=jax.ShapeDtypeStruct((ep, nfdim), f32),
        interpret=_IT,
    )(edge_attr, W_rbf1, row2(b_rbf1), W_rbf2, row2(b_rbf2))

    nblk = _blk(n, 2000)
    ngrid = n // nblk

    prev = nf0
    stats = None
    for l in range(l_layers):
        i32 = jnp.int32
        nq = hc // c
        hp = nq // 2
        wspecs = [
            pl.BlockSpec((nfdim, hc), lambda i: (0, 0)),
            pl.BlockSpec((1, hc), lambda i: (0, 0)),
        ] * 3
        out_shapes = (jax.ShapeDtypeStruct((n, nq, 128), i32),
                      jax.ShapeDtypeStruct((n, hp, 128), i32),
                      jax.ShapeDtypeStruct((n, nq, 128), i32))
        out_specs = (pl.BlockSpec((nblk, nq, 128), lambda i: (i, 0, 0)),
                     pl.BlockSpec((nblk, hp, 128), lambda i: (i, 0, 0)),
                     pl.BlockSpec((nblk, nq, 128), lambda i: (i, 0, 0)))
        wargs = (Wq[l], row2(bq[l]), Wk[l], row2(bk[l]), Wv[l], row2(bv[l]))
        if l == 0:
            qk_t, vd_t, kv_t = pl.pallas_call(
                functools.partial(_proj_kernel, hc=hc, c=c),
                grid=(ngrid,),
                in_specs=[pl.BlockSpec((nblk, nfdim), lambda i: (i, 0))] + wspecs,
                out_specs=out_specs,
                out_shape=out_shapes,
                interpret=_IT,
            )(prev, *wargs)
        else:
            qk_t, vd_t, kv_t = pl.pallas_call(
                functools.partial(_proj_bn_kernel, hc=hc, c=c, n_nodes=n),
                grid=(ngrid,),
                in_specs=[
                    pl.BlockSpec((nblk, nfdim), lambda i: (i, 0)),
                    pl.BlockSpec((2, nfdim), lambda i: (0, 0)),
                    pl.BlockSpec((1, nfdim), lambda i: (0, 0)),
                    pl.BlockSpec((1, nfdim), lambda i: (0, 0)),
                ] + wspecs,
                out_specs=out_specs,
                out_shape=out_shapes,
                interpret=_IT,
            )(prev, stats, row2(bn_g[l - 1]), row2(bn_b[l - 1]), *wargs)

        eblk2 = _blk(ep, 5120)
        e_t = pl.pallas_call(
            _eproj_kernel,
            grid=(ep // eblk2,),
            in_specs=[
                pl.BlockSpec((eblk2, nfdim), lambda i: (i, 0)),
                pl.BlockSpec((nfdim, hc), lambda i: (0, 0)),
            ],
            out_specs=pl.BlockSpec((eblk2, hc), lambda i: (i, 0)),
            out_shape=jax.ShapeDtypeStruct((ep, hc), f32),
            interpret=_IT,
        )(ef, We[l])

        g_qk = _make_gather(nq, ep, 64)(qk_t, dst)
        g_vd = _make_gather(hp, ep, 64)(vd_t, dst)
        g_kv = _make_gather(nq, ep, 64)(kv_t, src)

        eblk3 = _blk(ep, 1024)
        msg = pl.pallas_call(
            functools.partial(_edge_kernel, h=h, c=c),
            grid=(ep // eblk3,),
            in_specs=[
                pl.BlockSpec((eblk3, nq, 128), lambda i: (i, 0, 0)),
                pl.BlockSpec((eblk3, hp, 128), lambda i: (i, 0, 0)),
                pl.BlockSpec((eblk3, nq, 128), lambda i: (i, 0, 0)),
                pl.BlockSpec((eblk3, hc), lambda i: (i, 0)),
                pl.BlockSpec((3 * c, 3 * c), lambda i: (0, 0)),
                pl.BlockSpec((1, 3 * c), lambda i: (0, 0)),
                pl.BlockSpec((3 * c, c), lambda i: (0, 0)),
                pl.BlockSpec((1, c), lambda i: (0, 0)),
                pl.BlockSpec((1, 3 * c), lambda i: (0, 0)),
                pl.BlockSpec((1, 3 * c), lambda i: (0, 0)),
                pl.BlockSpec((1, c), lambda i: (0, 0)),
                pl.BlockSpec((1, c), lambda i: (0, 0)),
            ],
            out_specs=pl.BlockSpec((h, eblk3, c), lambda i: (0, i, 0)),
            out_shape=jax.ShapeDtypeStruct((h, ep, c), f32),
            interpret=_IT,
        )(g_qk, g_vd, g_kv, e_t, Wmu[l], row2(bmu[l]), Wml[l], row2(bml[l]),
          row2(ln_a_g[l]), row2(ln_a_b[l]), row2(ln_msg_g[l]),
          row2(ln_msg_b[l]))

        agg = _make_scatter(n, ep, h, c)(msg, dst_b)

        out_l, stats = pl.pallas_call(
            functools.partial(_update_kernel, h=h, c=c),
            grid=(ngrid,),
            in_specs=[
                pl.BlockSpec((h, nblk, c), lambda i: (0, i, 0)),
                pl.BlockSpec((hc, nfdim), lambda i: (0, 0)),
                pl.BlockSpec((1, nfdim), lambda i: (0, 0)),
            ],
            out_specs=(pl.BlockSpec((nblk, nfdim), lambda i: (i, 0)),
                       pl.BlockSpec((2, nfdim), lambda i: (0, 0))),
            out_shape=(jax.ShapeDtypeStruct((n, nfdim), f32),
                       jax.ShapeDtypeStruct((2, nfdim), f32)),
            interpret=_IT,
        )(agg, Wcat[l], row2(bcat[l]))
        prev = out_l

    out = pl.pallas_call(
        functools.partial(_pool_kernel, n_nodes=n),
        out_shape=jax.ShapeDtypeStruct((_G, 1), f32),
        interpret=_IT,
    )(prev, stats, row2(bn_g[l_layers - 1]), row2(bn_b[l_layers - 1]),
      batch.reshape(1, n), W_fc, row2(b_fc), W_out.reshape(1, -1),
      row2(b_out))
    return out.reshape(_G)


# speedup vs baseline: 1.0753x; 1.0753x over previous
import functools
import math

import jax
import jax.numpy as jnp
from jax import lax
from jax.experimental import pallas as pl
from jax.experimental.pallas import tpu as pltpu
from jax.experimental.pallas import tpu_sc as plsc

_IT = False
_G = 256
_EPS = 1e-5

_NSC = 2
_NTILE = 16


def _sigmoid(x):
    return 1.0 / (1.0 + jnp.exp(-x))


def _silu(x):
    return x * _sigmoid(x)


def _blk(n, target):
    if n % target == 0:
        return target
    return n


def _prologue_node_kernel(x_ref, w_ref, b_ref, o_ref):
    o_ref[...] = jnp.dot(x_ref[...], w_ref[...],
                         preferred_element_type=jnp.float32) + b_ref[...]


def _prologue_edge_kernel(ea_ref, w1_ref, b1_ref, w2_ref, b2_ref, o_ref, *,
                          efb):
    ea = ea_ref[...]
    d = jnp.sqrt(jnp.sum(ea * ea, axis=1, keepdims=True))
    step = 8.0 / (efb - 1)
    centers = lax.broadcasted_iota(jnp.int32, (1, efb), 1).astype(jnp.float32) * step
    gamma = 1.0 / (step * step)
    diff = d - centers
    rbf = jnp.exp(-gamma * diff * diff)
    z = jnp.dot(rbf, w1_ref[...], preferred_element_type=jnp.float32) + b1_ref[...]
    sp = jnp.maximum(z, 0.0) + jnp.log(1.0 + jnp.exp(-jnp.abs(z)))
    o_ref[...] = jnp.dot(sp, w2_ref[...],
                         preferred_element_type=jnp.float32) + b2_ref[...]


def _rne16(u):
    return u + jnp.uint32(0x7FFF) + ((u >> 16) & jnp.uint32(1))


def _pack2(lo, hi):
    ul = _rne16(lax.bitcast_convert_type(lo, jnp.uint32))
    uh = _rne16(lax.bitcast_convert_type(hi, jnp.uint32))
    packed = (uh & jnp.uint32(0xFFFF0000)) | (ul >> 16)
    return lax.bitcast_convert_type(packed, jnp.int32)


def _unpack(x, hi):
    u = lax.bitcast_convert_type(x, jnp.uint32)
    v = (u & jnp.uint32(0xFFFF0000)) if hi else (u << 16)
    return lax.bitcast_convert_type(v, jnp.float32)


def _store_tables(q, k, v, qk_ref, vd_ref, kv_ref, c):
    nq = q.shape[1] // c
    hp = nq // 2
    for t in range(hp):
        s0 = slice((2 * t) * c, (2 * t + 1) * c)
        s1 = slice((2 * t + 1) * c, (2 * t + 2) * c)
        qk_ref[:, t, :] = _pack2(q[:, s0], q[:, s1])
        qk_ref[:, hp + t, :] = _pack2(k[:, s0], k[:, s1])
        vd_ref[:, t, :] = _pack2(v[:, s0], v[:, s1])
        kv_ref[:, t, :] = _pack2(k[:, s0], k[:, s1])
        kv_ref[:, hp + t, :] = _pack2(v[:, s0], v[:, s1])


def _proj_kernel(p_ref, wq_ref, bq_ref, wk_ref, bk_ref, wv_ref, bv_ref,
                 qk_ref, vd_ref, kv_ref, *, hc, c):
    nf = p_ref[...]
    q = jnp.dot(nf, wq_ref[...], preferred_element_type=jnp.float32) + bq_ref[...]
    k = jnp.dot(nf, wk_ref[...], preferred_element_type=jnp.float32) + bk_ref[...]
    v = jnp.dot(nf, wv_ref[...], preferred_element_type=jnp.float32) + bv_ref[...]
    _store_tables(q, k, v, qk_ref, vd_ref, kv_ref, c)


def _proj_bn_kernel(p_ref, stats_ref, g_ref, b_ref,
                    wq_ref, bq_ref, wk_ref, bk_ref, wv_ref, bv_ref,
                    qk_ref, vd_ref, kv_ref, *, hc, c, n_nodes):
    p = p_ref[...]
    m = stats_ref[0:1, :] / n_nodes
    var = stats_ref[1:2, :] / n_nodes - m * m
    xb = (p - m) * lax.rsqrt(var + _EPS) * g_ref[...] + b_ref[...]
    nf = _silu(xb)
    q = jnp.dot(nf, wq_ref[...], preferred_element_type=jnp.float32) + bq_ref[...]
    k = jnp.dot(nf, wk_ref[...], preferred_element_type=jnp.float32) + bk_ref[...]
    v = jnp.dot(nf, wv_ref[...], preferred_element_type=jnp.float32) + bv_ref[...]
    _store_tables(q, k, v, qk_ref, vd_ref, kv_ref, c)


def _eproj_kernel(ef_ref, we_ref, o_ref):
    o_ref[...] = jnp.dot(ef_ref[...], we_ref[...],
                         preferred_element_type=jnp.float32)


def _edge_kernel(gqk_ref, gvd_ref, gkv_ref, e_ref, wmu_ref, bmu_ref,
                 wml_ref, bml_ref,
                 lag_ref, lab_ref, lmg_ref, lmb_ref, o_ref, *, h, c):
    scale = 1.0 / math.sqrt(3.0 * c)
    f32 = jnp.float32
    wmu = wmu_ref[...]
    bmu = bmu_ref[...]
    wml = wml_ref[...]
    bml = bml_ref[...]
    hp = h // 2
    for hh in range(h):
        sl = slice(hh * c, (hh + 1) * c)
        t, odd = hh // 2, hh % 2
        q = _unpack(gqk_ref[:, t, :], odd)
        k_i = _unpack(gqk_ref[:, hp + t, :], odd)
        v_i = _unpack(gvd_ref[:, t, :], odd)
        k_j = _unpack(gkv_ref[:, t, :], odd)
        v_j = _unpack(gkv_ref[:, hp + t, :], odd)
        e = e_ref[:, sl]
        a1 = q * k_i * scale
        a2 = q * k_j * scale
        a3 = q * e * scale
        s = (jnp.sum(a1, axis=1, keepdims=True)
             + jnp.sum(a2, axis=1, keepdims=True)
             + jnp.sum(a3, axis=1, keepdims=True))
        ss = (jnp.sum(a1 * a1, axis=1, keepdims=True)
              + jnp.sum(a2 * a2, axis=1, keepdims=True)
              + jnp.sum(a3 * a3, axis=1, keepdims=True))
        m = s / (3.0 * c)
        var = ss / (3.0 * c) - m * m
        r = lax.rsqrt(var + _EPS)
        g1 = _sigmoid((a1 - m) * r * lag_ref[:, 0:c] + lab_ref[:, 0:c])
        g2 = _sigmoid((a2 - m) * r * lag_ref[:, c:2 * c] + lab_ref[:, c:2 * c])
        g3 = _sigmoid((a3 - m) * r * lag_ref[:, 2 * c:3 * c] + lab_ref[:, 2 * c:3 * c])
        m2 = (jnp.dot(v_i, wmu[0:c, :], preferred_element_type=jnp.float32)
              + jnp.dot(v_j, wmu[c:2 * c, :], preferred_element_type=jnp.float32)
              + jnp.dot(e, wmu[2 * c:3 * c, :], preferred_element_type=jnp.float32)
              + bmu)
        m3 = (jnp.dot(m2[:, 0:c] * g1, wml[0:c, :],
                      preferred_element_type=jnp.float32)
              + jnp.dot(m2[:, c:2 * c] * g2, wml[c:2 * c, :],
                        preferred_element_type=jnp.float32)
              + jnp.dot(m2[:, 2 * c:3 * c] * g3, wml[2 * c:3 * c, :],
                        preferred_element_type=jnp.float32)
              + bml)
        mm = jnp.mean(m3, axis=1, keepdims=True)
        mv = jnp.mean(m3 * m3, axis=1, keepdims=True) - mm * mm
        msg = (m3 - mm) * lax.rsqrt(mv + _EPS) * lmg_ref[...] + lmb_ref[...]
        o_ref[hh, :, :] = msg


def _make_gather(sl, ep, b):
    per_tile = ep // (_NSC * _NTILE)
    nb = per_tile // b
    npair = nb // 2
    mesh = plsc.VectorSubcoreMesh(core_axis_name="c", subcore_axis_name="s")

    def body(table_hbm, idx_hbm, out_hbm, idx_v, r0, r1, g0, g1, o0, o1):
        cid = lax.axis_index("c")
        sid = lax.axis_index("s")
        wid = sid * _NSC + cid
        base = wid * per_tile
        pltpu.sync_copy(idx_hbm.at[pl.ds(base, per_tile)], idx_v)
        bufs = (r0, r1)
        gsem = (g0, g1)
        osem = (o0, o1)

        def start_g(j, p):
            pltpu.async_copy(table_hbm.at[idx_v.at[pl.ds(j * b, b)]],
                             bufs[p], gsem[p])

        def wait_g(j, p):
            pltpu.make_async_copy(table_hbm.at[idx_v.at[pl.ds(j * b, b)]],
                                  bufs[p], gsem[p]).wait()

        def start_o(j, p):
            pltpu.async_copy(bufs[p], out_hbm.at[pl.ds(base + j * b, b)],
                             osem[p])

        def wait_o(j, p):
            pltpu.make_async_copy(bufs[p],
                                  out_hbm.at[pl.ds(base + j * b, b)],
                                  osem[p]).wait()

        start_g(0, 0)
        start_g(1, 1)

        def pair(j2, carry):
            j = 2 * j2
            wait_g(j, 0)
            start_o(j, 0)
            wait_g(j + 1, 1)
            start_o(j + 1, 1)
            wait_o(j, 0)
            start_g(j + 2, 0)
            wait_o(j + 1, 1)
            start_g(j + 3, 1)
            return carry

        lax.fori_loop(0, npair - 1, pair, 0)
        j = nb - 2
        wait_g(j, 0)
        start_o(j, 0)
        wait_g(j + 1, 1)
        start_o(j + 1, 1)
        wait_o(j, 0)
        wait_o(j + 1, 1)

    def make(table, idx):
        return pl.kernel(
            body,
            out_type=jax.ShapeDtypeStruct((ep, sl, 128), jnp.int32),
            mesh=mesh,
            scratch_types=[
                pltpu.VMEM((per_tile,), jnp.int32),
                pltpu.VMEM((b, sl, 128), jnp.int32),
                pltpu.VMEM((b, sl, 128), jnp.int32),
                pltpu.SemaphoreType.DMA,
                pltpu.SemaphoreType.DMA,
                pltpu.SemaphoreType.DMA,
                pltpu.SemaphoreType.DMA,
            ],
        )(table, idx)

    return make


def _make_scatter(n, ep, h, c):
    ntab = ((n + _NTILE * 8) // (_NTILE * 8)) * (_NTILE * 8)
    br = 128
    per_tile_b = (ep // br) // _NTILE
    npair = per_tile_b // 2
    tab_slice = ntab // _NTILE
    rows_lo = (n // _NTILE) // 8 * 8
    rows_hi = n - rows_lo * (_NTILE - 1)
    vper = c // 16
    mesh = plsc.VectorSubcoreMesh(core_axis_name="c", subcore_axis_name="s")

    @functools.partial(
        pl.kernel,
        out_type=jax.ShapeDtypeStruct((h, n, c), jnp.float32),
        mesh=mesh,
        scratch_types=[
            pltpu.VMEM((128, c), jnp.float32),
            pltpu.VMEM((br, c), jnp.float32),
            pltpu.VMEM((br, c), jnp.float32),
            pltpu.VMEM((128,), jnp.int32),
            pltpu.VMEM((128,), jnp.int32),
            pltpu.VMEM((128,), jnp.int32),
            pltpu.VMEM((128,), jnp.int32),
            pltpu.VMEM_SHARED((ntab, c), jnp.float32),
            pltpu.SemaphoreType.DMA,
            pltpu.SemaphoreType.DMA,
        ],
    )
    def scat(msg_hbm, dstb_hbm, agg_hbm, zero_v, rows0, rows1,
             ia0, ib0, ia1, ib1, table, s0, s1):
        cid = lax.axis_index("c")
        sid = lax.axis_index("s")
        rows = (rows0, rows1)
        ia = (ia0, ia1)
        ib = (ib0, ib1)
        sem = (s0, s1)

        def zbody(i, carry):
            r = i // vper
            col = (i % vper) * 16
            zero_v[r, pl.ds(col, 16)] = jnp.zeros((16,), jnp.float32)
            return carry

        lax.fori_loop(0, 128 * vper, zbody, 0)
        tab_base = sid * tab_slice

        for ck in range(h // _NSC):
            chunk = cid * (h // _NSC) + ck
            off = 0
            left = tab_slice
            while left > 0:
                step = min(128, left)
                pltpu.sync_copy(zero_v.at[pl.ds(0, step)],
                                table.at[pl.ds(tab_base + off, step)])
                off += step
                left -= step
            plsc.subcore_barrier()

            def load(j, p):
                e0 = (sid * per_tile_b + j) * br
                pltpu.async_copy(dstb_hbm.at[pl.ds(e0, 128)], ia[p], sem[p])
                pltpu.async_copy(msg_hbm.at[chunk, pl.ds(e0, br)], rows[p],
                                 sem[p])

            def wait_load(j, p):
                e0 = (sid * per_tile_b + j) * br
                pltpu.make_async_copy(dstb_hbm.at[pl.ds(e0, 128)], ia[p],
                                      sem[p]).wait()
                pltpu.make_async_copy(msg_hbm.at[chunk, pl.ds(e0, br)],
                                      rows[p], sem[p]).wait()

            def add2(p):
                pltpu.sync_copy(rows[p], table.at[ia[p]], add=True)

            load(0, 0)

            def pair(j2, carry):
                j = 2 * j2
                load(j + 1, 1)
                wait_load(j, 0)
                add2(0)
                load(j + 2, 0)
                wait_load(j + 1, 1)
                add2(1)
                return carry

            lax.fori_loop(0, npair - 1, pair, 0)
            j = per_tile_b - 2
            load(j + 1, 1)
            wait_load(j, 0)
            add2(0)
            wait_load(j + 1, 1)
            add2(1)

            plsc.subcore_barrier()
            out_base = sid * rows_lo

            @pl.when(sid < _NTILE - 1)
            def _copy_lo():
                pltpu.sync_copy(table.at[pl.ds(out_base, rows_lo)],
                                agg_hbm.at[chunk, pl.ds(out_base, rows_lo)])

            @pl.when(sid == _NTILE - 1)
            def _copy_hi():
                base = rows_lo * (_NTILE - 1)
                pltpu.sync_copy(table.at[pl.ds(base, rows_hi)],
                                agg_hbm.at[chunk, pl.ds(base, rows_hi)])

            plsc.subcore_barrier()

    return scat


def _update_kernel(agg_ref, wcat_ref, bcat_ref, o_ref, stats_ref, *, h, c):
    i = pl.program_id(0)
    wcat = wcat_ref[...]
    o = bcat_ref[...]
    for hh in range(h):
        o = o + jnp.dot(agg_ref[hh, :, :], wcat[hh * c:(hh + 1) * c, :],
                        preferred_element_type=jnp.float32)
    o_ref[...] = o

    @pl.when(i == 0)
    def _init():
        stats_ref[...] = jnp.zeros_like(stats_ref)

    s = jnp.sum(o, axis=0, keepdims=True)
    ss = jnp.sum(o * o, axis=0, keepdims=True)
    stats_ref[0:1, :] = stats_ref[0:1, :] + s
    stats_ref[1:2, :] = stats_ref[1:2, :] + ss


def _pool_kernel(p_ref, stats_ref, g_ref, b_ref, batch_ref,
                 wfc_ref, bfc_ref, wout_ref, bout_ref, o_ref, *, n_nodes):
    p = p_ref[...]
    m = stats_ref[0:1, :] / n_nodes
    var = stats_ref[1:2, :] / n_nodes - m * m
    nf = _silu((p - m) * lax.rsqrt(var + _EPS) * g_ref[...] + b_ref[...])
    seg = batch_ref[...]
    gids = lax.broadcasted_iota(jnp.int32, (_G, 1), 0)
    onehot = (seg == gids).astype(jnp.float32)
    pooled = jnp.dot(onehot, nf, preferred_element_type=jnp.float32)
    counts = jnp.sum(onehot, axis=1, keepdims=True)
    pooled = pooled / jnp.maximum(counts, 1.0)
    feat = _silu(jnp.dot(pooled, wfc_ref[...],
                         preferred_element_type=jnp.float32) + bfc_ref[...])
    out = jnp.sum(feat * wout_ref[...], axis=1, keepdims=True) + bout_ref[...]
    o_ref[...] = out


def kernel(x, edge_attr, edge_index, batch, W_atom, b_atom, W_rbf1, b_rbf1,
           W_rbf2, b_rbf2, Wq, bq, Wk, bk, Wv, bv, We, Wcat, bcat, Wmu, bmu,
           Wml, bml, ln_msg_g, ln_msg_b, ln_a_g, ln_a_b, bn_g, bn_b,
           W_fc, b_fc, W_out, b_out):
    n, aif = x.shape
    e = edge_attr.shape[0]
    nfdim = W_atom.shape[1]
    efb = W_rbf1.shape[0]
    l_layers = Wq.shape[0]
    c = Wcat.shape[2]
    h = Wq.shape[2] // c
    hc = h * c

    ep = ((e + 4095) // 4096) * 4096
    pad = ep - e
    src = jnp.concatenate([edge_index[0], jnp.zeros((pad,), jnp.int32)])
    dst = jnp.concatenate([edge_index[1], jnp.zeros((pad,), jnp.int32)])
    dst_b = jnp.concatenate(
        [edge_index[1], jnp.full((pad,), n, jnp.int32)])
    edge_attr = jnp.concatenate(
        [edge_attr, jnp.zeros((pad, edge_attr.shape[1]), edge_attr.dtype)])

    f32 = jnp.float32
    row2 = lambda a: a.reshape(1, -1)

    nf0 = pl.pallas_call(
        _prologue_node_kernel,
        out_shape=jax.ShapeDtypeStruct((n, nfdim), f32),
        interpret=_IT,
    )(x, W_atom, row2(b_atom))

    eblk = _blk(ep, 5120)
    ef = pl.pallas_call(
        functools.partial(_prologue_edge_kernel, efb=efb),
        grid=(ep // eblk,),
        in_specs=[
            pl.BlockSpec((eblk, edge_attr.shape[1]), lambda i: (i, 0)),
            pl.BlockSpec((efb, nfdim), lambda i: (0, 0)),
            pl.BlockSpec((1, nfdim), lambda i: (0, 0)),
            pl.BlockSpec((nfdim, nfdim), lambda i: (0, 0)),
            pl.BlockSpec((1, nfdim), lambda i: (0, 0)),
        ],
        out_specs=pl.BlockSpec((eblk, nfdim), lambda i: (i, 0)),
        out_shape=jax.ShapeDtypeStruct((ep, nfdim), f32),
        interpret=_IT,
    )(edge_attr, W_rbf1, row2(b_rbf1), W_rbf2, row2(b_rbf2))

    nblk = _blk(n, 2000)
    ngrid = n // nblk

    prev = nf0
    stats = None
    for l in range(l_layers):
        i32 = jnp.int32
        nq = hc // c
        hp = nq // 2
        wspecs = [
            pl.BlockSpec((nfdim, hc), lambda i: (0, 0)),
            pl.BlockSpec((1, hc), lambda i: (0, 0)),
        ] * 3
        out_shapes = (jax.ShapeDtypeStruct((n, nq, 128), i32),
                      jax.ShapeDtypeStruct((n, hp, 128), i32),
                      jax.ShapeDtypeStruct((n, nq, 128), i32))
        out_specs = (pl.BlockSpec((nblk, nq, 128), lambda i: (i, 0, 0)),
                     pl.BlockSpec((nblk, hp, 128), lambda i: (i, 0, 0)),
                     pl.BlockSpec((nblk, nq, 128), lambda i: (i, 0, 0)))
        wargs = (Wq[l], row2(bq[l]), Wk[l], row2(bk[l]), Wv[l], row2(bv[l]))
        if l == 0:
            qk_t, vd_t, kv_t = pl.pallas_call(
                functools.partial(_proj_kernel, hc=hc, c=c),
                grid=(ngrid,),
                in_specs=[pl.BlockSpec((nblk, nfdim), lambda i: (i, 0))] + wspecs,
                out_specs=out_specs,
                out_shape=out_shapes,
                interpret=_IT,
            )(prev, *wargs)
        else:
            qk_t, vd_t, kv_t = pl.pallas_call(
                functools.partial(_proj_bn_kernel, hc=hc, c=c, n_nodes=n),
                grid=(ngrid,),
                in_specs=[
                    pl.BlockSpec((nblk, nfdim), lambda i: (i, 0)),
                    pl.BlockSpec((2, nfdim), lambda i: (0, 0)),
                    pl.BlockSpec((1, nfdim), lambda i: (0, 0)),
                    pl.BlockSpec((1, nfdim), lambda i: (0, 0)),
                ] + wspecs,
                out_specs=out_specs,
                out_shape=out_shapes,
                interpret=_IT,
            )(prev, stats, row2(bn_g[l - 1]), row2(bn_b[l - 1]), *wargs)

        eblk2 = _blk(ep, 5120)
        e_t = pl.pallas_call(
            _eproj_kernel,
            grid=(ep // eblk2,),
            in_specs=[
                pl.BlockSpec((eblk2, nfdim), lambda i: (i, 0)),
                pl.BlockSpec((nfdim, hc), lambda i: (0, 0)),
            ],
            out_specs=pl.BlockSpec((eblk2, hc), lambda i: (i, 0)),
            out_shape=jax.ShapeDtypeStruct((ep, hc), f32),
            interpret=_IT,
        )(ef, We[l])

        g_qk = _make_gather(nq, ep, 64)(qk_t, dst)
        g_vd = _make_gather(hp, ep, 64)(vd_t, dst)
        g_kv = _make_gather(nq, ep, 64)(kv_t, src)

        eblk3 = _blk(ep, 1024)
        msg = pl.pallas_call(
            functools.partial(_edge_kernel, h=h, c=c),
            grid=(ep // eblk3,),
            in_specs=[
                pl.BlockSpec((eblk3, nq, 128), lambda i: (i, 0, 0)),
                pl.BlockSpec((eblk3, hp, 128), lambda i: (i, 0, 0)),
                pl.BlockSpec((eblk3, nq, 128), lambda i: (i, 0, 0)),
                pl.BlockSpec((eblk3, hc), lambda i: (i, 0)),
                pl.BlockSpec((3 * c, 3 * c), lambda i: (0, 0)),
                pl.BlockSpec((1, 3 * c), lambda i: (0, 0)),
                pl.BlockSpec((3 * c, c), lambda i: (0, 0)),
                pl.BlockSpec((1, c), lambda i: (0, 0)),
                pl.BlockSpec((1, 3 * c), lambda i: (0, 0)),
                pl.BlockSpec((1, 3 * c), lambda i: (0, 0)),
                pl.BlockSpec((1, c), lambda i: (0, 0)),
                pl.BlockSpec((1, c), lambda i: (0, 0)),
            ],
            out_specs=pl.BlockSpec((h, eblk3, c), lambda i: (0, i, 0)),
            out_shape=jax.ShapeDtypeStruct((h, ep, c), f32),
            interpret=_IT,
        )(g_qk, g_vd, g_kv, e_t, Wmu[l], row2(bmu[l]), Wml[l], row2(bml[l]),
          row2(ln_a_g[l]), row2(ln_a_b[l]), row2(ln_msg_g[l]),
          row2(ln_msg_b[l]))

        agg = _make_scatter(n, ep, h, c)(msg, dst_b)

        out_l, stats = pl.pallas_call(
            functools.partial(_update_kernel, h=h, c=c),
            grid=(ngrid,),
            in_specs=[
                pl.BlockSpec((h, nblk, c), lambda i: (0, i, 0)),
                pl.BlockSpec((hc, nfdim), lambda i: (0, 0)),
                pl.BlockSpec((1, nfdim), lambda i: (0, 0)),
            ],
            out_specs=(pl.BlockSpec((nblk, nfdim), lambda i: (i, 0)),
                       pl.BlockSpec((2, nfdim), lambda i: (0, 0))),
            out_shape=(jax.ShapeDtypeStruct((n, nfdim), f32),
                       jax.ShapeDtypeStruct((2, nfdim), f32)),
            interpret=_IT,
        )(agg, Wcat[l], row2(bcat[l]))
        prev = out_l

    out = pl.pallas_call(
        functools.partial(_pool_kernel, n_nodes=n),
        out_shape=jax.ShapeDtypeStruct((_G, 1), f32),
        interpret=_IT,
    )(prev, stats, row2(bn_g[l_layers - 1]), row2(bn_b[l_layers - 1]),
      batch.reshape(1, n), W_fc, row2(b_fc), W_out.reshape(1, -1),
      row2(b_out))
    return out.reshape(_G)


# speedup vs baseline: 3.3843x; 3.1472x over previous
import functools
import math

import jax
import jax.numpy as jnp
from jax import lax
from jax.experimental import pallas as pl
from jax.experimental.pallas import tpu as pltpu
from jax.experimental.pallas import tpu_sc as plsc

_IT = False
_G = 256
_EPS = 1e-5

_NSC = 2
_NTILE = 16


def _sigmoid(x):
    return 1.0 / (1.0 + jnp.exp(-x))


def _silu(x):
    return x * _sigmoid(x)


def _blk(n, target):
    if n % target == 0:
        return target
    return n


def _prologue_node_kernel(x_ref, w_ref, b_ref, o_ref):
    o_ref[...] = jnp.dot(x_ref[...], w_ref[...],
                         preferred_element_type=jnp.float32) + b_ref[...]


def _prologue_edge_kernel(ea_ref, w1_ref, b1_ref, w2_ref, b2_ref, o_ref, *,
                          efb):
    ea = ea_ref[...]
    d = jnp.sqrt(jnp.sum(ea * ea, axis=1, keepdims=True))
    step = 8.0 / (efb - 1)
    centers = lax.broadcasted_iota(jnp.int32, (1, efb), 1).astype(jnp.float32) * step
    gamma = 1.0 / (step * step)
    diff = d - centers
    rbf = jnp.exp(-gamma * diff * diff)
    z = jnp.dot(rbf, w1_ref[...], preferred_element_type=jnp.float32) + b1_ref[...]
    sp = jnp.maximum(z, 0.0) + jnp.log(1.0 + jnp.exp(-jnp.abs(z)))
    o_ref[...] = jnp.dot(sp, w2_ref[...],
                         preferred_element_type=jnp.float32) + b2_ref[...]


def _rne16(u):
    return u + jnp.uint32(0x7FFF) + ((u >> 16) & jnp.uint32(1))


def _pack2(lo, hi):
    ul = _rne16(lax.bitcast_convert_type(lo, jnp.uint32))
    uh = _rne16(lax.bitcast_convert_type(hi, jnp.uint32))
    packed = (uh & jnp.uint32(0xFFFF0000)) | (ul >> 16)
    return lax.bitcast_convert_type(packed, jnp.int32)


def _unpack(x, hi):
    u = lax.bitcast_convert_type(x, jnp.uint32)
    v = (u & jnp.uint32(0xFFFF0000)) if hi else (u << 16)
    return lax.bitcast_convert_type(v, jnp.float32)


def _store_tables(q, k, v, qk_ref, vd_ref, kv_ref, c):
    nq = q.shape[1] // c
    hp = nq // 2
    for t in range(hp):
        s0 = slice((2 * t) * c, (2 * t + 1) * c)
        s1 = slice((2 * t + 1) * c, (2 * t + 2) * c)
        d = slice(t * c, (t + 1) * c)
        d2 = slice((hp + t) * c, (hp + t + 1) * c)
        kp = _pack2(k[:, s0], k[:, s1])
        vp = _pack2(v[:, s0], v[:, s1])
        qk_ref[:, d] = _pack2(q[:, s0], q[:, s1])
        qk_ref[:, d2] = kp
        vd_ref[:, d] = vp
        kv_ref[:, d] = kp
        kv_ref[:, d2] = vp


def _proj_kernel(p_ref, wq_ref, bq_ref, wk_ref, bk_ref, wv_ref, bv_ref,
                 qk_ref, vd_ref, kv_ref, *, hc, c):
    nf = p_ref[...]
    q = jnp.dot(nf, wq_ref[...], preferred_element_type=jnp.float32) + bq_ref[...]
    k = jnp.dot(nf, wk_ref[...], preferred_element_type=jnp.float32) + bk_ref[...]
    v = jnp.dot(nf, wv_ref[...], preferred_element_type=jnp.float32) + bv_ref[...]
    _store_tables(q, k, v, qk_ref, vd_ref, kv_ref, c)


def _proj_bn_kernel(p_ref, stats_ref, g_ref, b_ref,
                    wq_ref, bq_ref, wk_ref, bk_ref, wv_ref, bv_ref,
                    qk_ref, vd_ref, kv_ref, *, hc, c, n_nodes):
    p = p_ref[...]
    m = stats_ref[0:1, :] / n_nodes
    var = stats_ref[1:2, :] / n_nodes - m * m
    xb = (p - m) * lax.rsqrt(var + _EPS) * g_ref[...] + b_ref[...]
    nf = _silu(xb)
    q = jnp.dot(nf, wq_ref[...], preferred_element_type=jnp.float32) + bq_ref[...]
    k = jnp.dot(nf, wk_ref[...], preferred_element_type=jnp.float32) + bk_ref[...]
    v = jnp.dot(nf, wv_ref[...], preferred_element_type=jnp.float32) + bv_ref[...]
    _store_tables(q, k, v, qk_ref, vd_ref, kv_ref, c)


def _eproj_kernel(ef_ref, we_ref, o_ref):
    o_ref[...] = jnp.dot(ef_ref[...], we_ref[...],
                         preferred_element_type=jnp.float32)


def _edge_kernel(gqk_ref, gvd_ref, gkv_ref, e_ref, wmu_ref, bmu_ref,
                 wml_ref, bml_ref,
                 lag_ref, lab_ref, lmg_ref, lmb_ref, o_ref, *, h, c):
    scale = 1.0 / math.sqrt(3.0 * c)
    f32 = jnp.float32
    wmu = wmu_ref[...]
    bmu = bmu_ref[...]
    wml = wml_ref[...]
    bml = bml_ref[...]
    hp = h // 2
    for hh in range(h):
        sl = slice(hh * c, (hh + 1) * c)
        t, odd = hh // 2, hh % 2
        ts = slice(t * c, (t + 1) * c)
        t2 = slice((hp + t) * c, (hp + t + 1) * c)
        q = _unpack(gqk_ref[:, ts], odd)
        k_i = _unpack(gqk_ref[:, t2], odd)
        v_i = _unpack(gvd_ref[:, ts], odd)
        k_j = _unpack(gkv_ref[:, ts], odd)
        v_j = _unpack(gkv_ref[:, t2], odd)
        e = e_ref[:, sl]
        a1 = q * k_i * scale
        a2 = q * k_j * scale
        a3 = q * e * scale
        s = (jnp.sum(a1, axis=1, keepdims=True)
             + jnp.sum(a2, axis=1, keepdims=True)
             + jnp.sum(a3, axis=1, keepdims=True))
        ss = (jnp.sum(a1 * a1, axis=1, keepdims=True)
              + jnp.sum(a2 * a2, axis=1, keepdims=True)
              + jnp.sum(a3 * a3, axis=1, keepdims=True))
        m = s / (3.0 * c)
        var = ss / (3.0 * c) - m * m
        r = lax.rsqrt(var + _EPS)
        g1 = _sigmoid((a1 - m) * r * lag_ref[:, 0:c] + lab_ref[:, 0:c])
        g2 = _sigmoid((a2 - m) * r * lag_ref[:, c:2 * c] + lab_ref[:, c:2 * c])
        g3 = _sigmoid((a3 - m) * r * lag_ref[:, 2 * c:3 * c] + lab_ref[:, 2 * c:3 * c])
        m2 = (jnp.dot(v_i, wmu[0:c, :], preferred_element_type=jnp.float32)
              + jnp.dot(v_j, wmu[c:2 * c, :], preferred_element_type=jnp.float32)
              + jnp.dot(e, wmu[2 * c:3 * c, :], preferred_element_type=jnp.float32)
              + bmu)
        m3 = (jnp.dot(m2[:, 0:c] * g1, wml[0:c, :],
                      preferred_element_type=jnp.float32)
              + jnp.dot(m2[:, c:2 * c] * g2, wml[c:2 * c, :],
                        preferred_element_type=jnp.float32)
              + jnp.dot(m2[:, 2 * c:3 * c] * g3, wml[2 * c:3 * c, :],
                        preferred_element_type=jnp.float32)
              + bml)
        mm = jnp.mean(m3, axis=1, keepdims=True)
        mv = jnp.mean(m3 * m3, axis=1, keepdims=True) - mm * mm
        msg = (m3 - mm) * lax.rsqrt(mv + _EPS) * lmg_ref[...] + lmb_ref[...]
        o_ref[hh, :, :] = msg


def _make_gather(d, ep, b):
    per_tile = ep // (_NSC * _NTILE)
    nb = per_tile // b
    npair = nb // 2
    mesh = plsc.VectorSubcoreMesh(core_axis_name="c", subcore_axis_name="s")

    def body(table_hbm, idx_hbm, out_hbm, idx_v, r0, r1, g0, g1, o0, o1):
        cid = lax.axis_index("c")
        sid = lax.axis_index("s")
        wid = sid * _NSC + cid
        base = wid * per_tile
        pltpu.sync_copy(idx_hbm.at[pl.ds(base, per_tile)], idx_v)
        bufs = (r0, r1)
        gsem = (g0, g1)
        osem = (o0, o1)

        def start_g(j, p):
            pltpu.async_copy(table_hbm.at[idx_v.at[pl.ds(j * b, b)]],
                             bufs[p], gsem[p])

        def wait_g(j, p):
            pltpu.make_async_copy(table_hbm.at[idx_v.at[pl.ds(j * b, b)]],
                                  bufs[p], gsem[p]).wait()

        def start_o(j, p):
            pltpu.async_copy(bufs[p], out_hbm.at[pl.ds(base + j * b, b)],
                             osem[p])

        def wait_o(j, p):
            pltpu.make_async_copy(bufs[p],
                                  out_hbm.at[pl.ds(base + j * b, b)],
                                  osem[p]).wait()

        start_g(0, 0)
        start_g(1, 1)

        def pair(j2, carry):
            j = 2 * j2
            wait_g(j, 0)
            start_o(j, 0)
            wait_g(j + 1, 1)
            start_o(j + 1, 1)
            wait_o(j, 0)
            start_g(j + 2, 0)
            wait_o(j + 1, 1)
            start_g(j + 3, 1)
            return carry

        lax.fori_loop(0, npair - 1, pair, 0)
        j = nb - 2
        wait_g(j, 0)
        start_o(j, 0)
        wait_g(j + 1, 1)
        start_o(j + 1, 1)
        wait_o(j, 0)
        wait_o(j + 1, 1)

    def make(table, idx):
        return pl.kernel(
            body,
            out_type=jax.ShapeDtypeStruct((ep, d), jnp.int32),
            mesh=mesh,
            scratch_types=[
                pltpu.VMEM((per_tile,), jnp.int32),
                pltpu.VMEM((b, d), jnp.int32),
                pltpu.VMEM((b, d), jnp.int32),
                pltpu.SemaphoreType.DMA,
                pltpu.SemaphoreType.DMA,
                pltpu.SemaphoreType.DMA,
                pltpu.SemaphoreType.DMA,
            ],
        )(table, idx)

    return make


def _make_scatter(n, ep, h, c):
    ntab = ((n + _NTILE * 8) // (_NTILE * 8)) * (_NTILE * 8)
    br = 128
    per_tile_b = (ep // br) // _NTILE
    npair = per_tile_b // 2
    tab_slice = ntab // _NTILE
    rows_lo = (n // _NTILE) // 8 * 8
    rows_hi = n - rows_lo * (_NTILE - 1)
    vper = c // 16
    mesh = plsc.VectorSubcoreMesh(core_axis_name="c", subcore_axis_name="s")

    @functools.partial(
        pl.kernel,
        out_type=jax.ShapeDtypeStruct((h, n, c), jnp.float32),
        mesh=mesh,
        scratch_types=[
            pltpu.VMEM((128, c), jnp.float32),
            pltpu.VMEM((br, c), jnp.float32),
            pltpu.VMEM((br, c), jnp.float32),
            pltpu.VMEM((128,), jnp.int32),
            pltpu.VMEM((128,), jnp.int32),
            pltpu.VMEM((128,), jnp.int32),
            pltpu.VMEM((128,), jnp.int32),
            pltpu.VMEM_SHARED((ntab, c), jnp.float32),
            pltpu.SemaphoreType.DMA,
            pltpu.SemaphoreType.DMA,
        ],
    )
    def scat(msg_hbm, dstb_hbm, agg_hbm, zero_v, rows0, rows1,
             ia0, ib0, ia1, ib1, table, s0, s1):
        cid = lax.axis_index("c")
        sid = lax.axis_index("s")
        rows = (rows0, rows1)
        ia = (ia0, ia1)
        ib = (ib0, ib1)
        sem = (s0, s1)

        def zbody(i, carry):
            r = i // vper
            col = (i % vper) * 16
            zero_v[r, pl.ds(col, 16)] = jnp.zeros((16,), jnp.float32)
            return carry

        lax.fori_loop(0, 128 * vper, zbody, 0)
        tab_base = sid * tab_slice

        for ck in range(h // _NSC):
            chunk = cid * (h // _NSC) + ck
            off = 0
            left = tab_slice
            while left > 0:
                step = min(128, left)
                pltpu.sync_copy(zero_v.at[pl.ds(0, step)],
                                table.at[pl.ds(tab_base + off, step)])
                off += step
                left -= step
            plsc.subcore_barrier()

            def load(j, p):
                e0 = (sid * per_tile_b + j) * br
                pltpu.async_copy(dstb_hbm.at[pl.ds(e0, 128)], ia[p], sem[p])
                pltpu.async_copy(msg_hbm.at[chunk, pl.ds(e0, br)], rows[p],
                                 sem[p])

            def wait_load(j, p):
                e0 = (sid * per_tile_b + j) * br
                pltpu.make_async_copy(dstb_hbm.at[pl.ds(e0, 128)], ia[p],
                                      sem[p]).wait()
                pltpu.make_async_copy(msg_hbm.at[chunk, pl.ds(e0, br)],
                                      rows[p], sem[p]).wait()

            def add2(p):
                pltpu.sync_copy(rows[p], table.at[ia[p]], add=True)

            load(0, 0)

            def pair(j2, carry):
                j = 2 * j2
                load(j + 1, 1)
                wait_load(j, 0)
                add2(0)
                load(j + 2, 0)
                wait_load(j + 1, 1)
                add2(1)
                return carry

            lax.fori_loop(0, npair - 1, pair, 0)
            j = per_tile_b - 2
            load(j + 1, 1)
            wait_load(j, 0)
            add2(0)
            wait_load(j + 1, 1)
            add2(1)

            plsc.subcore_barrier()
            out_base = sid * rows_lo

            @pl.when(sid < _NTILE - 1)
            def _copy_lo():
                pltpu.sync_copy(table.at[pl.ds(out_base, rows_lo)],
                                agg_hbm.at[chunk, pl.ds(out_base, rows_lo)])

            @pl.when(sid == _NTILE - 1)
            def _copy_hi():
                base = rows_lo * (_NTILE - 1)
                pltpu.sync_copy(table.at[pl.ds(base, rows_hi)],
                                agg_hbm.at[chunk, pl.ds(base, rows_hi)])

            plsc.subcore_barrier()

    return scat


def _update_kernel(agg_ref, wcat_ref, bcat_ref, o_ref, stats_ref, *, h, c):
    i = pl.program_id(0)
    wcat = wcat_ref[...]
    o = bcat_ref[...]
    for hh in range(h):
        o = o + jnp.dot(agg_ref[hh, :, :], wcat[hh * c:(hh + 1) * c, :],
                        preferred_element_type=jnp.float32)
    o_ref[...] = o

    @pl.when(i == 0)
    def _init():
        stats_ref[...] = jnp.zeros_like(stats_ref)

    s = jnp.sum(o, axis=0, keepdims=True)
    ss = jnp.sum(o * o, axis=0, keepdims=True)
    stats_ref[0:1, :] = stats_ref[0:1, :] + s
    stats_ref[1:2, :] = stats_ref[1:2, :] + ss


def _pool_kernel(p_ref, stats_ref, g_ref, b_ref, batch_ref,
                 wfc_ref, bfc_ref, wout_ref, bout_ref, o_ref, *, n_nodes):
    p = p_ref[...]
    m = stats_ref[0:1, :] / n_nodes
    var = stats_ref[1:2, :] / n_nodes - m * m
    nf = _silu((p - m) * lax.rsqrt(var + _EPS) * g_ref[...] + b_ref[...])
    seg = batch_ref[...]
    gids = lax.broadcasted_iota(jnp.int32, (_G, 1), 0)
    onehot = (seg == gids).astype(jnp.float32)
    pooled = jnp.dot(onehot, nf, preferred_element_type=jnp.float32)
    counts = jnp.sum(onehot, axis=1, keepdims=True)
    pooled = pooled / jnp.maximum(counts, 1.0)
    feat = _silu(jnp.dot(pooled, wfc_ref[...],
                         preferred_element_type=jnp.float32) + bfc_ref[...])
    out = jnp.sum(feat * wout_ref[...], axis=1, keepdims=True) + bout_ref[...]
    o_ref[...] = out


def kernel(x, edge_attr, edge_index, batch, W_atom, b_atom, W_rbf1, b_rbf1,
           W_rbf2, b_rbf2, Wq, bq, Wk, bk, Wv, bv, We, Wcat, bcat, Wmu, bmu,
           Wml, bml, ln_msg_g, ln_msg_b, ln_a_g, ln_a_b, bn_g, bn_b,
           W_fc, b_fc, W_out, b_out):
    n, aif = x.shape
    e = edge_attr.shape[0]
    nfdim = W_atom.shape[1]
    efb = W_rbf1.shape[0]
    l_layers = Wq.shape[0]
    c = Wcat.shape[2]
    h = Wq.shape[2] // c
    hc = h * c

    ep = ((e + 4095) // 4096) * 4096
    pad = ep - e
    src = jnp.concatenate([edge_index[0], jnp.zeros((pad,), jnp.int32)])
    dst = jnp.concatenate([edge_index[1], jnp.zeros((pad,), jnp.int32)])
    dst_b = jnp.concatenate(
        [edge_index[1], jnp.full((pad,), n, jnp.int32)])
    edge_attr = jnp.concatenate(
        [edge_attr, jnp.zeros((pad, edge_attr.shape[1]), edge_attr.dtype)])

    f32 = jnp.float32
    row2 = lambda a: a.reshape(1, -1)

    nf0 = pl.pallas_call(
        _prologue_node_kernel,
        out_shape=jax.ShapeDtypeStruct((n, nfdim), f32),
        interpret=_IT,
    )(x, W_atom, row2(b_atom))

    eblk = _blk(ep, 5120)
    ef = pl.pallas_call(
        functools.partial(_prologue_edge_kernel, efb=efb),
        grid=(ep // eblk,),
        in_specs=[
            pl.BlockSpec((eblk, edge_attr.shape[1]), lambda i: (i, 0)),
            pl.BlockSpec((efb, nfdim), lambda i: (0, 0)),
            pl.BlockSpec((1, nfdim), lambda i: (0, 0)),
            pl.BlockSpec((nfdim, nfdim), lambda i: (0, 0)),
            pl.BlockSpec((1, nfdim), lambda i: (0, 0)),
        ],
        out_specs=pl.BlockSpec((eblk, nfdim), lambda i: (i, 0)),
        out_shape=jax.ShapeDtypeStruct((ep, nfdim), f32),
        interpret=_IT,
    )(edge_attr, W_rbf1, row2(b_rbf1), W_rbf2, row2(b_rbf2))

    nblk = _blk(n, 2000)
    ngrid = n // nblk

    prev = nf0
    stats = None
    for l in range(l_layers):
        i32 = jnp.int32
        nq = hc // c
        hp = nq // 2
        wspecs = [
            pl.BlockSpec((nfdim, hc), lambda i: (0, 0)),
            pl.BlockSpec((1, hc), lambda i: (0, 0)),
        ] * 3
        out_shapes = (jax.ShapeDtypeStruct((n, nq * 128), i32),
                      jax.ShapeDtypeStruct((n, hp * 128), i32),
                      jax.ShapeDtypeStruct((n, nq * 128), i32))
        out_specs = (pl.BlockSpec((nblk, nq * 128), lambda i: (i, 0)),
                     pl.BlockSpec((nblk, hp * 128), lambda i: (i, 0)),
                     pl.BlockSpec((nblk, nq * 128), lambda i: (i, 0)))
        wargs = (Wq[l], row2(bq[l]), Wk[l], row2(bk[l]), Wv[l], row2(bv[l]))
        if l == 0:
            qk_t, vd_t, kv_t = pl.pallas_call(
                functools.partial(_proj_kernel, hc=hc, c=c),
                grid=(ngrid,),
                in_specs=[pl.BlockSpec((nblk, nfdim), lambda i: (i, 0))] + wspecs,
                out_specs=out_specs,
                out_shape=out_shapes,
                interpret=_IT,
            )(prev, *wargs)
        else:
            qk_t, vd_t, kv_t = pl.pallas_call(
                functools.partial(_proj_bn_kernel, hc=hc, c=c, n_nodes=n),
                grid=(ngrid,),
                in_specs=[
                    pl.BlockSpec((nblk, nfdim), lambda i: (i, 0)),
                    pl.BlockSpec((2, nfdim), lambda i: (0, 0)),
                    pl.BlockSpec((1, nfdim), lambda i: (0, 0)),
                    pl.BlockSpec((1, nfdim), lambda i: (0, 0)),
                ] + wspecs,
                out_specs=out_specs,
                out_shape=out_shapes,
                interpret=_IT,
            )(prev, stats, row2(bn_g[l - 1]), row2(bn_b[l - 1]), *wargs)

        eblk2 = _blk(ep, 5120)
        e_t = pl.pallas_call(
            _eproj_kernel,
            grid=(ep // eblk2,),
            in_specs=[
                pl.BlockSpec((eblk2, nfdim), lambda i: (i, 0)),
                pl.BlockSpec((nfdim, hc), lambda i: (0, 0)),
            ],
            out_specs=pl.BlockSpec((eblk2, hc), lambda i: (i, 0)),
            out_shape=jax.ShapeDtypeStruct((ep, hc), f32),
            interpret=_IT,
        )(ef, We[l])

        g_qk = _make_gather(nq * 128, ep, 64)(qk_t, dst)
        g_vd = _make_gather(hp * 128, ep, 64)(vd_t, dst)
        g_kv = _make_gather(nq * 128, ep, 64)(kv_t, src)

        eblk3 = _blk(ep, 1024)
        msg = pl.pallas_call(
            functools.partial(_edge_kernel, h=h, c=c),
            grid=(ep // eblk3,),
            in_specs=[
                pl.BlockSpec((eblk3, nq * 128), lambda i: (i, 0)),
                pl.BlockSpec((eblk3, hp * 128), lambda i: (i, 0)),
                pl.BlockSpec((eblk3, nq * 128), lambda i: (i, 0)),
                pl.BlockSpec((eblk3, hc), lambda i: (i, 0)),
                pl.BlockSpec((3 * c, 3 * c), lambda i: (0, 0)),
                pl.BlockSpec((1, 3 * c), lambda i: (0, 0)),
                pl.BlockSpec((3 * c, c), lambda i: (0, 0)),
                pl.BlockSpec((1, c), lambda i: (0, 0)),
                pl.BlockSpec((1, 3 * c), lambda i: (0, 0)),
                pl.BlockSpec((1, 3 * c), lambda i: (0, 0)),
                pl.BlockSpec((1, c), lambda i: (0, 0)),
                pl.BlockSpec((1, c), lambda i: (0, 0)),
            ],
            out_specs=pl.BlockSpec((h, eblk3, c), lambda i: (0, i, 0)),
            out_shape=jax.ShapeDtypeStruct((h, ep, c), f32),
            interpret=_IT,
        )(g_qk, g_vd, g_kv, e_t, Wmu[l], row2(bmu[l]), Wml[l], row2(bml[l]),
          row2(ln_a_g[l]), row2(ln_a_b[l]), row2(ln_msg_g[l]),
          row2(ln_msg_b[l]))

        agg = _make_scatter(n, ep, h, c)(msg, dst_b)

        out_l, stats = pl.pallas_call(
            functools.partial(_update_kernel, h=h, c=c),
            grid=(ngrid,),
            in_specs=[
                pl.BlockSpec((h, nblk, c), lambda i: (0, i, 0)),
                pl.BlockSpec((hc, nfdim), lambda i: (0, 0)),
                pl.BlockSpec((1, nfdim), lambda i: (0, 0)),
            ],
            out_specs=(pl.BlockSpec((nblk, nfdim), lambda i: (i, 0)),
                       pl.BlockSpec((2, nfdim), lambda i: (0, 0))),
            out_shape=(jax.ShapeDtypeStruct((n, nfdim), f32),
                       jax.ShapeDtypeStruct((2, nfdim), f32)),
            interpret=_IT,
        )(agg, Wcat[l], row2(bcat[l]))
        prev = out_l

    out = pl.pallas_call(
        functools.partial(_pool_kernel, n_nodes=n),
        out_shape=jax.ShapeDtypeStruct((_G, 1), f32),
        interpret=_IT,
    )(prev, stats, row2(bn_g[l_layers - 1]), row2(bn_b[l_layers - 1]),
      batch.reshape(1, n), W_fc, row2(b_fc), W_out.reshape(1, -1),
      row2(b_out))
    return out.reshape(_G)


# speedup vs baseline: 3.6792x; 1.0871x over previous
import functools
import math

import jax
import jax.numpy as jnp
from jax import lax
from jax.experimental import pallas as pl
from jax.experimental.pallas import tpu as pltpu
from jax.experimental.pallas import tpu_sc as plsc

_IT = False
_G = 256
_EPS = 1e-5

_NSC = 2
_NTILE = 16


def _sigmoid(x):
    return 1.0 / (1.0 + jnp.exp(-x))


def _silu(x):
    return x * _sigmoid(x)


def _blk(n, target):
    if n % target == 0:
        return target
    return n


def _prologue_node_kernel(x_ref, w_ref, b_ref, o_ref):
    o_ref[...] = jnp.dot(x_ref[...], w_ref[...],
                         preferred_element_type=jnp.float32) + b_ref[...]


def _prologue_edge_kernel(ea_ref, w1_ref, b1_ref, w2_ref, b2_ref, o_ref, *,
                          efb):
    ea = ea_ref[...]
    d = jnp.sqrt(jnp.sum(ea * ea, axis=1, keepdims=True))
    step = 8.0 / (efb - 1)
    centers = lax.broadcasted_iota(jnp.int32, (1, efb), 1).astype(jnp.float32) * step
    gamma = 1.0 / (step * step)
    diff = d - centers
    rbf = jnp.exp(-gamma * diff * diff)
    z = jnp.dot(rbf, w1_ref[...], preferred_element_type=jnp.float32) + b1_ref[...]
    sp = jnp.maximum(z, 0.0) + jnp.log(1.0 + jnp.exp(-jnp.abs(z)))
    o_ref[...] = jnp.dot(sp, w2_ref[...],
                         preferred_element_type=jnp.float32) + b2_ref[...]


def _rne16(u):
    return u + jnp.uint32(0x7FFF) + ((u >> 16) & jnp.uint32(1))


def _pack2(lo, hi):
    ul = _rne16(lax.bitcast_convert_type(lo, jnp.uint32))
    uh = _rne16(lax.bitcast_convert_type(hi, jnp.uint32))
    packed = (uh & jnp.uint32(0xFFFF0000)) | (ul >> 16)
    return lax.bitcast_convert_type(packed, jnp.int32)


def _unpack(x, hi):
    u = lax.bitcast_convert_type(x, jnp.uint32)
    v = (u & jnp.uint32(0xFFFF0000)) if hi else (u << 16)
    return lax.bitcast_convert_type(v, jnp.float32)


def _store_tables(q, k, v, qkv_ref, kv_ref, c):
    nq = q.shape[1] // c
    hp = nq // 2
    for t in range(hp):
        s0 = slice((2 * t) * c, (2 * t + 1) * c)
        s1 = slice((2 * t + 1) * c, (2 * t + 2) * c)
        d = slice(t * c, (t + 1) * c)
        d2 = slice((hp + t) * c, (hp + t + 1) * c)
        d3 = slice((2 * hp + t) * c, (2 * hp + t + 1) * c)
        kp = _pack2(k[:, s0], k[:, s1])
        vp = _pack2(v[:, s0], v[:, s1])
        qkv_ref[:, d] = _pack2(q[:, s0], q[:, s1])
        qkv_ref[:, d2] = kp
        qkv_ref[:, d3] = vp
        kv_ref[:, d] = kp
        kv_ref[:, d2] = vp


def _proj_kernel(p_ref, wq_ref, bq_ref, wk_ref, bk_ref, wv_ref, bv_ref,
                 qkv_ref, kv_ref, *, hc, c):
    nf = p_ref[...]
    q = jnp.dot(nf, wq_ref[...], preferred_element_type=jnp.float32) + bq_ref[...]
    k = jnp.dot(nf, wk_ref[...], preferred_element_type=jnp.float32) + bk_ref[...]
    v = jnp.dot(nf, wv_ref[...], preferred_element_type=jnp.float32) + bv_ref[...]
    _store_tables(q, k, v, qkv_ref, kv_ref, c)


def _proj_bn_kernel(p_ref, stats_ref, g_ref, b_ref,
                    wq_ref, bq_ref, wk_ref, bk_ref, wv_ref, bv_ref,
                    qkv_ref, kv_ref, *, hc, c, n_nodes):
    p = p_ref[...]
    m = stats_ref[0:1, :] / n_nodes
    var = stats_ref[1:2, :] / n_nodes - m * m
    xb = (p - m) * lax.rsqrt(var + _EPS) * g_ref[...] + b_ref[...]
    nf = _silu(xb)
    q = jnp.dot(nf, wq_ref[...], preferred_element_type=jnp.float32) + bq_ref[...]
    k = jnp.dot(nf, wk_ref[...], preferred_element_type=jnp.float32) + bk_ref[...]
    v = jnp.dot(nf, wv_ref[...], preferred_element_type=jnp.float32) + bv_ref[...]
    _store_tables(q, k, v, qkv_ref, kv_ref, c)


def _eproj_kernel(ef_ref, we_ref, o_ref):
    o_ref[...] = jnp.dot(ef_ref[...], we_ref[...],
                         preferred_element_type=jnp.float32)


def _edge_kernel(gqkv_ref, gkv_ref, e_ref, wmu_ref, bmu_ref,
                 wml_ref, bml_ref,
                 lag_ref, lab_ref, lmg_ref, lmb_ref, o_ref, *, h, c):
    scale = 1.0 / math.sqrt(3.0 * c)
    f32 = jnp.float32
    wmu = wmu_ref[...]
    bmu = bmu_ref[...]
    wml = wml_ref[...]
    bml = bml_ref[...]
    hp = h // 2
    for hh in range(h):
        sl = slice(hh * c, (hh + 1) * c)
        t, odd = hh // 2, hh % 2
        ts = slice(t * c, (t + 1) * c)
        t2 = slice((hp + t) * c, (hp + t + 1) * c)
        t3 = slice((2 * hp + t) * c, (2 * hp + t + 1) * c)
        q = _unpack(gqkv_ref[:, ts], odd)
        k_i = _unpack(gqkv_ref[:, t2], odd)
        v_i = _unpack(gqkv_ref[:, t3], odd)
        k_j = _unpack(gkv_ref[:, ts], odd)
        v_j = _unpack(gkv_ref[:, t2], odd)
        e = e_ref[:, sl]
        a1 = q * k_i * scale
        a2 = q * k_j * scale
        a3 = q * e * scale
        s = (jnp.sum(a1, axis=1, keepdims=True)
             + jnp.sum(a2, axis=1, keepdims=True)
             + jnp.sum(a3, axis=1, keepdims=True))
        ss = (jnp.sum(a1 * a1, axis=1, keepdims=True)
              + jnp.sum(a2 * a2, axis=1, keepdims=True)
              + jnp.sum(a3 * a3, axis=1, keepdims=True))
        m = s / (3.0 * c)
        var = ss / (3.0 * c) - m * m
        r = lax.rsqrt(var + _EPS)
        g1 = _sigmoid((a1 - m) * r * lag_ref[:, 0:c] + lab_ref[:, 0:c])
        g2 = _sigmoid((a2 - m) * r * lag_ref[:, c:2 * c] + lab_ref[:, c:2 * c])
        g3 = _sigmoid((a3 - m) * r * lag_ref[:, 2 * c:3 * c] + lab_ref[:, 2 * c:3 * c])
        m2 = (jnp.dot(v_i, wmu[0:c, :], preferred_element_type=jnp.float32)
              + jnp.dot(v_j, wmu[c:2 * c, :], preferred_element_type=jnp.float32)
              + jnp.dot(e, wmu[2 * c:3 * c, :], preferred_element_type=jnp.float32)
              + bmu)
        m3 = (jnp.dot(m2[:, 0:c] * g1, wml[0:c, :],
                      preferred_element_type=jnp.float32)
              + jnp.dot(m2[:, c:2 * c] * g2, wml[c:2 * c, :],
                        preferred_element_type=jnp.float32)
              + jnp.dot(m2[:, 2 * c:3 * c] * g3, wml[2 * c:3 * c, :],
                        preferred_element_type=jnp.float32)
              + bml)
        mm = jnp.mean(m3, axis=1, keepdims=True)
        mv = jnp.mean(m3 * m3, axis=1, keepdims=True) - mm * mm
        msg = (m3 - mm) * lax.rsqrt(mv + _EPS) * lmg_ref[...] + lmb_ref[...]
        o_ref[hh, :, :] = msg


def _make_gather(d, ep, b):
    per_tile = ep // (_NSC * _NTILE)
    nb = per_tile // b
    npair = nb // 2
    mesh = plsc.VectorSubcoreMesh(core_axis_name="c", subcore_axis_name="s")

    def body(table_hbm, idx_hbm, out_hbm, idx_v, r0, r1, g0, g1, o0, o1):
        cid = lax.axis_index("c")
        sid = lax.axis_index("s")
        wid = sid * _NSC + cid
        base = wid * per_tile
        pltpu.sync_copy(idx_hbm.at[pl.ds(base, per_tile)], idx_v)
        bufs = (r0, r1)
        gsem = (g0, g1)
        osem = (o0, o1)

        def start_g(j, p):
            pltpu.async_copy(table_hbm.at[idx_v.at[pl.ds(j * b, b)]],
                             bufs[p], gsem[p])

        def wait_g(j, p):
            pltpu.make_async_copy(table_hbm.at[idx_v.at[pl.ds(j * b, b)]],
                                  bufs[p], gsem[p]).wait()

        def start_o(j, p):
            pltpu.async_copy(bufs[p], out_hbm.at[pl.ds(base + j * b, b)],
                             osem[p])

        def wait_o(j, p):
            pltpu.make_async_copy(bufs[p],
                                  out_hbm.at[pl.ds(base + j * b, b)],
                                  osem[p]).wait()

        start_g(0, 0)
        start_g(1, 1)

        def pair(j2, carry):
            j = 2 * j2
            wait_g(j, 0)
            start_o(j, 0)
            wait_g(j + 1, 1)
            start_o(j + 1, 1)
            wait_o(j, 0)
            start_g(j + 2, 0)
            wait_o(j + 1, 1)
            start_g(j + 3, 1)
            return carry

        lax.fori_loop(0, npair - 1, pair, 0)
        j = nb - 2
        wait_g(j, 0)
        start_o(j, 0)
        wait_g(j + 1, 1)
        start_o(j + 1, 1)
        wait_o(j, 0)
        wait_o(j + 1, 1)

    def make(table, idx):
        return pl.kernel(
            body,
            out_type=jax.ShapeDtypeStruct((ep, d), jnp.int32),
            mesh=mesh,
            scratch_types=[
                pltpu.VMEM((per_tile,), jnp.int32),
                pltpu.VMEM((b, d), jnp.int32),
                pltpu.VMEM((b, d), jnp.int32),
                pltpu.SemaphoreType.DMA,
                pltpu.SemaphoreType.DMA,
                pltpu.SemaphoreType.DMA,
                pltpu.SemaphoreType.DMA,
            ],
        )(table, idx)

    return make


def _make_scatter(n, ep, h, c):
    ntab = ((n + _NTILE * 8) // (_NTILE * 8)) * (_NTILE * 8)
    br = 128
    per_tile_b = (ep // br) // _NTILE
    npair = per_tile_b // 2
    tab_slice = ntab // _NTILE
    rows_lo = (n // _NTILE) // 8 * 8
    rows_hi = n - rows_lo * (_NTILE - 1)
    vper = c // 16
    mesh = plsc.VectorSubcoreMesh(core_axis_name="c", subcore_axis_name="s")

    @functools.partial(
        pl.kernel,
        out_type=jax.ShapeDtypeStruct((h, n, c), jnp.float32),
        mesh=mesh,
        scratch_types=[
            pltpu.VMEM((128, c), jnp.float32),
            pltpu.VMEM((br, c), jnp.float32),
            pltpu.VMEM((br, c), jnp.float32),
            pltpu.VMEM((128,), jnp.int32),
            pltpu.VMEM((128,), jnp.int32),
            pltpu.VMEM((128,), jnp.int32),
            pltpu.VMEM((128,), jnp.int32),
            pltpu.VMEM_SHARED((ntab, c), jnp.float32),
            pltpu.SemaphoreType.DMA,
            pltpu.SemaphoreType.DMA,
        ],
    )
    def scat(msg_hbm, dstb_hbm, agg_hbm, zero_v, rows0, rows1,
             ia0, ib0, ia1, ib1, table, s0, s1):
        cid = lax.axis_index("c")
        sid = lax.axis_index("s")
        rows = (rows0, rows1)
        ia = (ia0, ia1)
        ib = (ib0, ib1)
        sem = (s0, s1)

        def zbody(i, carry):
            r = i // vper
            col = (i % vper) * 16
            zero_v[r, pl.ds(col, 16)] = jnp.zeros((16,), jnp.float32)
            return carry

        lax.fori_loop(0, 128 * vper, zbody, 0)
        tab_base = sid * tab_slice

        for ck in range(h // _NSC):
            chunk = cid * (h // _NSC) + ck
            off = 0
            left = tab_slice
            while left > 0:
                step = min(128, left)
                pltpu.sync_copy(zero_v.at[pl.ds(0, step)],
                                table.at[pl.ds(tab_base + off, step)])
                off += step
                left -= step
            plsc.subcore_barrier()

            def load(j, p):
                e0 = (sid * per_tile_b + j) * br
                pltpu.async_copy(dstb_hbm.at[pl.ds(e0, 128)], ia[p], sem[p])
                pltpu.async_copy(msg_hbm.at[chunk, pl.ds(e0, br)], rows[p],
                                 sem[p])

            def wait_load(j, p):
                e0 = (sid * per_tile_b + j) * br
                pltpu.make_async_copy(dstb_hbm.at[pl.ds(e0, 128)], ia[p],
                                      sem[p]).wait()
                pltpu.make_async_copy(msg_hbm.at[chunk, pl.ds(e0, br)],
                                      rows[p], sem[p]).wait()

            def add2(p):
                pltpu.sync_copy(rows[p], table.at[ia[p]], add=True)

            load(0, 0)

            def pair(j2, carry):
                j = 2 * j2
                load(j + 1, 1)
                wait_load(j, 0)
                add2(0)
                load(j + 2, 0)
                wait_load(j + 1, 1)
                add2(1)
                return carry

            lax.fori_loop(0, npair - 1, pair, 0)
            j = per_tile_b - 2
            load(j + 1, 1)
            wait_load(j, 0)
            add2(0)
            wait_load(j + 1, 1)
            add2(1)

            plsc.subcore_barrier()
            out_base = sid * rows_lo

            @pl.when(sid < _NTILE - 1)
            def _copy_lo():
                pltpu.sync_copy(table.at[pl.ds(out_base, rows_lo)],
                                agg_hbm.at[chunk, pl.ds(out_base, rows_lo)])

            @pl.when(sid == _NTILE - 1)
            def _copy_hi():
                base = rows_lo * (_NTILE - 1)
                pltpu.sync_copy(table.at[pl.ds(base, rows_hi)],
                                agg_hbm.at[chunk, pl.ds(base, rows_hi)])

            plsc.subcore_barrier()

    return scat


def _update_kernel(agg_ref, wcat_ref, bcat_ref, o_ref, stats_ref, *, h, c):
    i = pl.program_id(0)
    wcat = wcat_ref[...]
    o = bcat_ref[...]
    for hh in range(h):
        o = o + jnp.dot(agg_ref[hh, :, :], wcat[hh * c:(hh + 1) * c, :],
                        preferred_element_type=jnp.float32)
    o_ref[...] = o

    @pl.when(i == 0)
    def _init():
        stats_ref[...] = jnp.zeros_like(stats_ref)

    s = jnp.sum(o, axis=0, keepdims=True)
    ss = jnp.sum(o * o, axis=0, keepdims=True)
    stats_ref[0:1, :] = stats_ref[0:1, :] + s
    stats_ref[1:2, :] = stats_ref[1:2, :] + ss


def _pool_kernel(p_ref, stats_ref, g_ref, b_ref, batch_ref,
                 wfc_ref, bfc_ref, wout_ref, bout_ref, o_ref, *, n_nodes):
    p = p_ref[...]
    m = stats_ref[0:1, :] / n_nodes
    var = stats_ref[1:2, :] / n_nodes - m * m
    nf = _silu((p - m) * lax.rsqrt(var + _EPS) * g_ref[...] + b_ref[...])
    seg = batch_ref[...]
    gids = lax.broadcasted_iota(jnp.int32, (_G, 1), 0)
    onehot = (seg == gids).astype(jnp.float32)
    pooled = jnp.dot(onehot, nf, preferred_element_type=jnp.float32)
    counts = jnp.sum(onehot, axis=1, keepdims=True)
    pooled = pooled / jnp.maximum(counts, 1.0)
    feat = _silu(jnp.dot(pooled, wfc_ref[...],
                         preferred_element_type=jnp.float32) + bfc_ref[...])
    out = jnp.sum(feat * wout_ref[...], axis=1, keepdims=True) + bout_ref[...]
    o_ref[...] = out


def kernel(x, edge_attr, edge_index, batch, W_atom, b_atom, W_rbf1, b_rbf1,
           W_rbf2, b_rbf2, Wq, bq, Wk, bk, Wv, bv, We, Wcat, bcat, Wmu, bmu,
           Wml, bml, ln_msg_g, ln_msg_b, ln_a_g, ln_a_b, bn_g, bn_b,
           W_fc, b_fc, W_out, b_out):
    n, aif = x.shape
    e = edge_attr.shape[0]
    nfdim = W_atom.shape[1]
    efb = W_rbf1.shape[0]
    l_layers = Wq.shape[0]
    c = Wcat.shape[2]
    h = Wq.shape[2] // c
    hc = h * c

    ep = ((e + 4095) // 4096) * 4096
    pad = ep - e
    src = jnp.concatenate([edge_index[0], jnp.zeros((pad,), jnp.int32)])
    dst = jnp.concatenate([edge_index[1], jnp.zeros((pad,), jnp.int32)])
    dst_b = jnp.concatenate(
        [edge_index[1], jnp.full((pad,), n, jnp.int32)])
    edge_attr = jnp.concatenate(
        [edge_attr, jnp.zeros((pad, edge_attr.shape[1]), edge_attr.dtype)])

    f32 = jnp.float32
    row2 = lambda a: a.reshape(1, -1)

    nf0 = pl.pallas_call(
        _prologue_node_kernel,
        out_shape=jax.ShapeDtypeStruct((n, nfdim), f32),
        interpret=_IT,
    )(x, W_atom, row2(b_atom))

    eblk = _blk(ep, 5120)
    ef = pl.pallas_call(
        functools.partial(_prologue_edge_kernel, efb=efb),
        grid=(ep // eblk,),
        in_specs=[
            pl.BlockSpec((eblk, edge_attr.shape[1]), lambda i: (i, 0)),
            pl.BlockSpec((efb, nfdim), lambda i: (0, 0)),
            pl.BlockSpec((1, nfdim), lambda i: (0, 0)),
            pl.BlockSpec((nfdim, nfdim), lambda i: (0, 0)),
            pl.BlockSpec((1, nfdim), lambda i: (0, 0)),
        ],
        out_specs=pl.BlockSpec((eblk, nfdim), lambda i: (i, 0)),
        out_shape=jax.ShapeDtypeStruct((ep, nfdim), f32),
        interpret=_IT,
    )(edge_attr, W_rbf1, row2(b_rbf1), W_rbf2, row2(b_rbf2))

    nblk = _blk(n, 2000)
    ngrid = n // nblk

    prev = nf0
    stats = None
    for l in range(l_layers):
        i32 = jnp.int32
        nq = hc // c
        hp = nq // 2
        wspecs = [
            pl.BlockSpec((nfdim, hc), lambda i: (0, 0)),
            pl.BlockSpec((1, hc), lambda i: (0, 0)),
        ] * 3
        out_shapes = (jax.ShapeDtypeStruct((n, 3 * hp * 128), i32),
                      jax.ShapeDtypeStruct((n, nq * 128), i32))
        out_specs = (pl.BlockSpec((nblk, 3 * hp * 128), lambda i: (i, 0)),
                     pl.BlockSpec((nblk, nq * 128), lambda i: (i, 0)))
        wargs = (Wq[l], row2(bq[l]), Wk[l], row2(bk[l]), Wv[l], row2(bv[l]))
        if l == 0:
            qkv_t, kv_t = pl.pallas_call(
                functools.partial(_proj_kernel, hc=hc, c=c),
                grid=(ngrid,),
                in_specs=[pl.BlockSpec((nblk, nfdim), lambda i: (i, 0))] + wspecs,
                out_specs=out_specs,
                out_shape=out_shapes,
                interpret=_IT,
            )(prev, *wargs)
        else:
            qkv_t, kv_t = pl.pallas_call(
                functools.partial(_proj_bn_kernel, hc=hc, c=c, n_nodes=n),
                grid=(ngrid,),
                in_specs=[
                    pl.BlockSpec((nblk, nfdim), lambda i: (i, 0)),
                    pl.BlockSpec((2, nfdim), lambda i: (0, 0)),
                    pl.BlockSpec((1, nfdim), lambda i: (0, 0)),
                    pl.BlockSpec((1, nfdim), lambda i: (0, 0)),
                ] + wspecs,
                out_specs=out_specs,
                out_shape=out_shapes,
                interpret=_IT,
            )(prev, stats, row2(bn_g[l - 1]), row2(bn_b[l - 1]), *wargs)

        eblk2 = _blk(ep, 5120)
        e_t = pl.pallas_call(
            _eproj_kernel,
            grid=(ep // eblk2,),
            in_specs=[
                pl.BlockSpec((eblk2, nfdim), lambda i: (i, 0)),
                pl.BlockSpec((nfdim, hc), lambda i: (0, 0)),
            ],
            out_specs=pl.BlockSpec((eblk2, hc), lambda i: (i, 0)),
            out_shape=jax.ShapeDtypeStruct((ep, hc), f32),
            interpret=_IT,
        )(ef, We[l])

        g_qkv = _make_gather(3 * hp * 128, ep, 64)(qkv_t, dst)
        g_kv = _make_gather(nq * 128, ep, 64)(kv_t, src)

        eblk3 = _blk(ep, 1024)
        msg = pl.pallas_call(
            functools.partial(_edge_kernel, h=h, c=c),
            grid=(ep // eblk3,),
            in_specs=[
                pl.BlockSpec((eblk3, 3 * hp * 128), lambda i: (i, 0)),
                pl.BlockSpec((eblk3, nq * 128), lambda i: (i, 0)),
                pl.BlockSpec((eblk3, hc), lambda i: (i, 0)),
                pl.BlockSpec((3 * c, 3 * c), lambda i: (0, 0)),
                pl.BlockSpec((1, 3 * c), lambda i: (0, 0)),
                pl.BlockSpec((3 * c, c), lambda i: (0, 0)),
                pl.BlockSpec((1, c), lambda i: (0, 0)),
                pl.BlockSpec((1, 3 * c), lambda i: (0, 0)),
                pl.BlockSpec((1, 3 * c), lambda i: (0, 0)),
                pl.BlockSpec((1, c), lambda i: (0, 0)),
                pl.BlockSpec((1, c), lambda i: (0, 0)),
            ],
            out_specs=pl.BlockSpec((h, eblk3, c), lambda i: (0, i, 0)),
            out_shape=jax.ShapeDtypeStruct((h, ep, c), f32),
            interpret=_IT,
        )(g_qkv, g_kv, e_t, Wmu[l], row2(bmu[l]), Wml[l], row2(bml[l]),
          row2(ln_a_g[l]), row2(ln_a_b[l]), row2(ln_msg_g[l]),
          row2(ln_msg_b[l]))

        agg = _make_scatter(n, ep, h, c)(msg, dst_b)

        out_l, stats = pl.pallas_call(
            functools.partial(_update_kernel, h=h, c=c),
            grid=(ngrid,),
            in_specs=[
                pl.BlockSpec((h, nblk, c), lambda i: (0, i, 0)),
                pl.BlockSpec((hc, nfdim), lambda i: (0, 0)),
                pl.BlockSpec((1, nfdim), lambda i: (0, 0)),
            ],
            out_specs=(pl.BlockSpec((nblk, nfdim), lambda i: (i, 0)),
                       pl.BlockSpec((2, nfdim), lambda i: (0, 0))),
            out_shape=(jax.ShapeDtypeStruct((n, nfdim), f32),
                       jax.ShapeDtypeStruct((2, nfdim), f32)),
            interpret=_IT,
        )(agg, Wcat[l], row2(bcat[l]))
        prev = out_l

    out = pl.pallas_call(
        functools.partial(_pool_kernel, n_nodes=n),
        out_shape=jax.ShapeDtypeStruct((_G, 1), f32),
        interpret=_IT,
    )(prev, stats, row2(bn_g[l_layers - 1]), row2(bn_b[l_layers - 1]),
      batch.reshape(1, n), W_fc, row2(b_fc), W_out.reshape(1, -1),
      row2(b_out))
    return out.reshape(_G)


# speedup vs baseline: 3.6878x; 1.0023x over previous
import functools
import math

import jax
import jax.numpy as jnp
from jax import lax
from jax.experimental import pallas as pl
from jax.experimental.pallas import tpu as pltpu
from jax.experimental.pallas import tpu_sc as plsc

_IT = False
_G = 256
_EPS = 1e-5

_NSC = 2
_NTILE = 16


def _sigmoid(x):
    return 1.0 / (1.0 + jnp.exp(-x))


def _silu(x):
    return x * _sigmoid(x)


def _blk(n, target):
    if n % target == 0:
        return target
    return n


def _prologue_node_kernel(x_ref, w_ref, b_ref, o_ref):
    o_ref[...] = jnp.dot(x_ref[...], w_ref[...],
                         preferred_element_type=jnp.float32) + b_ref[...]


def _prologue_edge_kernel(ea_ref, w1_ref, b1_ref, w2_ref, b2_ref, o_ref, *,
                          efb):
    ea = ea_ref[...]
    d = jnp.sqrt(jnp.sum(ea * ea, axis=1, keepdims=True))
    step = 8.0 / (efb - 1)
    centers = lax.broadcasted_iota(jnp.int32, (1, efb), 1).astype(jnp.float32) * step
    gamma = 1.0 / (step * step)
    diff = d - centers
    rbf = jnp.exp(-gamma * diff * diff)
    z = jnp.dot(rbf, w1_ref[...], preferred_element_type=jnp.float32) + b1_ref[...]
    sp = jnp.maximum(z, 0.0) + jnp.log(1.0 + jnp.exp(-jnp.abs(z)))
    o_ref[...] = jnp.dot(sp, w2_ref[...],
                         preferred_element_type=jnp.float32) + b2_ref[...]


def _rne16(u):
    return u + jnp.uint32(0x7FFF) + ((u >> 16) & jnp.uint32(1))


def _pack2(lo, hi):
    ul = _rne16(lax.bitcast_convert_type(lo, jnp.uint32))
    uh = _rne16(lax.bitcast_convert_type(hi, jnp.uint32))
    packed = (uh & jnp.uint32(0xFFFF0000)) | (ul >> 16)
    return lax.bitcast_convert_type(packed, jnp.int32)


def _unpack(x, hi):
    u = lax.bitcast_convert_type(x, jnp.uint32)
    v = (u & jnp.uint32(0xFFFF0000)) if hi else (u << 16)
    return lax.bitcast_convert_type(v, jnp.float32)


def _store_tables(q, k, v, qkv_ref, kv_ref, c):
    nq = q.shape[1] // c
    hp = nq // 2
    for t in range(hp):
        s0 = slice((2 * t) * c, (2 * t + 1) * c)
        s1 = slice((2 * t + 1) * c, (2 * t + 2) * c)
        d = slice(t * c, (t + 1) * c)
        d2 = slice((hp + t) * c, (hp + t + 1) * c)
        d3 = slice((2 * hp + t) * c, (2 * hp + t + 1) * c)
        kp = _pack2(k[:, s0], k[:, s1])
        vp = _pack2(v[:, s0], v[:, s1])
        qkv_ref[:, d] = _pack2(q[:, s0], q[:, s1])
        qkv_ref[:, d2] = kp
        qkv_ref[:, d3] = vp
        kv_ref[:, d] = kp
        kv_ref[:, d2] = vp


def _proj_kernel(p_ref, wq_ref, bq_ref, wk_ref, bk_ref, wv_ref, bv_ref,
                 qkv_ref, kv_ref, *, hc, c):
    nf = p_ref[...]
    q = jnp.dot(nf, wq_ref[...], preferred_element_type=jnp.float32) + bq_ref[...]
    k = jnp.dot(nf, wk_ref[...], preferred_element_type=jnp.float32) + bk_ref[...]
    v = jnp.dot(nf, wv_ref[...], preferred_element_type=jnp.float32) + bv_ref[...]
    _store_tables(q, k, v, qkv_ref, kv_ref, c)


def _proj_bn_kernel(p_ref, stats_ref, g_ref, b_ref,
                    wq_ref, bq_ref, wk_ref, bk_ref, wv_ref, bv_ref,
                    qkv_ref, kv_ref, *, hc, c, n_nodes):
    p = p_ref[...]
    m = stats_ref[0:1, :] / n_nodes
    var = stats_ref[1:2, :] / n_nodes - m * m
    xb = (p - m) * lax.rsqrt(var + _EPS) * g_ref[...] + b_ref[...]
    nf = _silu(xb)
    q = jnp.dot(nf, wq_ref[...], preferred_element_type=jnp.float32) + bq_ref[...]
    k = jnp.dot(nf, wk_ref[...], preferred_element_type=jnp.float32) + bk_ref[...]
    v = jnp.dot(nf, wv_ref[...], preferred_element_type=jnp.float32) + bv_ref[...]
    _store_tables(q, k, v, qkv_ref, kv_ref, c)


def _eproj_kernel(ef_ref, we_ref, o_ref):
    o_ref[...] = jnp.dot(ef_ref[...], we_ref[...],
                         preferred_element_type=jnp.float32)


def _edge_kernel(gqkv_ref, gkv_ref, e_ref, wmu_ref, bmu_ref,
                 wml_ref, bml_ref,
                 lag_ref, lab_ref, lmg_ref, lmb_ref, o_ref, *, h, c):
    scale = 1.0 / math.sqrt(3.0 * c)
    f32 = jnp.float32
    wmu = wmu_ref[...]
    bmu = bmu_ref[...]
    wml = wml_ref[...]
    bml = bml_ref[...]
    hp = h // 2
    for hh in range(h):
        sl = slice(hh * c, (hh + 1) * c)
        t, odd = hh // 2, hh % 2
        ts = slice(t * c, (t + 1) * c)
        t2 = slice((hp + t) * c, (hp + t + 1) * c)
        t3 = slice((2 * hp + t) * c, (2 * hp + t + 1) * c)
        q = _unpack(gqkv_ref[:, ts], odd)
        k_i = _unpack(gqkv_ref[:, t2], odd)
        v_i = _unpack(gqkv_ref[:, t3], odd)
        k_j = _unpack(gkv_ref[:, ts], odd)
        v_j = _unpack(gkv_ref[:, t2], odd)
        e = e_ref[:, sl]
        a1 = q * k_i * scale
        a2 = q * k_j * scale
        a3 = q * e * scale
        s = (jnp.sum(a1, axis=1, keepdims=True)
             + jnp.sum(a2, axis=1, keepdims=True)
             + jnp.sum(a3, axis=1, keepdims=True))
        ss = (jnp.sum(a1 * a1, axis=1, keepdims=True)
              + jnp.sum(a2 * a2, axis=1, keepdims=True)
              + jnp.sum(a3 * a3, axis=1, keepdims=True))
        m = s / (3.0 * c)
        var = ss / (3.0 * c) - m * m
        r = lax.rsqrt(var + _EPS)
        g1 = _sigmoid((a1 - m) * r * lag_ref[:, 0:c] + lab_ref[:, 0:c])
        g2 = _sigmoid((a2 - m) * r * lag_ref[:, c:2 * c] + lab_ref[:, c:2 * c])
        g3 = _sigmoid((a3 - m) * r * lag_ref[:, 2 * c:3 * c] + lab_ref[:, 2 * c:3 * c])
        m2 = (jnp.dot(v_i, wmu[0:c, :], preferred_element_type=jnp.float32)
              + jnp.dot(v_j, wmu[c:2 * c, :], preferred_element_type=jnp.float32)
              + jnp.dot(e, wmu[2 * c:3 * c, :], preferred_element_type=jnp.float32)
              + bmu)
        m3 = (jnp.dot(m2[:, 0:c] * g1, wml[0:c, :],
                      preferred_element_type=jnp.float32)
              + jnp.dot(m2[:, c:2 * c] * g2, wml[c:2 * c, :],
                        preferred_element_type=jnp.float32)
              + jnp.dot(m2[:, 2 * c:3 * c] * g3, wml[2 * c:3 * c, :],
                        preferred_element_type=jnp.float32)
              + bml)
        mm = jnp.mean(m3, axis=1, keepdims=True)
        mv = jnp.mean(m3 * m3, axis=1, keepdims=True) - mm * mm
        msg = (m3 - mm) * lax.rsqrt(mv + _EPS) * lmg_ref[...] + lmb_ref[...]
        o_ref[hh, :, :] = msg


def _make_gather(d, ep, b):
    per_tile = ep // (_NSC * _NTILE)
    nb = per_tile // b
    npair = nb // 2
    mesh = plsc.VectorSubcoreMesh(core_axis_name="c", subcore_axis_name="s")

    def body(table_hbm, idx_hbm, out_hbm, idx_v, r0, r1, g0, g1, o0, o1):
        cid = lax.axis_index("c")
        sid = lax.axis_index("s")
        wid = sid * _NSC + cid
        base = wid * per_tile
        pltpu.sync_copy(idx_hbm.at[pl.ds(base, per_tile)], idx_v)
        bufs = (r0, r1)
        gsem = (g0, g1)
        osem = (o0, o1)

        def start_g(j, p):
            pltpu.async_copy(table_hbm.at[idx_v.at[pl.ds(j * b, b)]],
                             bufs[p], gsem[p])

        def wait_g(j, p):
            pltpu.make_async_copy(table_hbm.at[idx_v.at[pl.ds(j * b, b)]],
                                  bufs[p], gsem[p]).wait()

        def start_o(j, p):
            pltpu.async_copy(bufs[p], out_hbm.at[pl.ds(base + j * b, b)],
                             osem[p])

        def wait_o(j, p):
            pltpu.make_async_copy(bufs[p],
                                  out_hbm.at[pl.ds(base + j * b, b)],
                                  osem[p]).wait()

        start_g(0, 0)
        start_g(1, 1)

        def pair(j2, carry):
            j = 2 * j2
            wait_g(j, 0)
            start_o(j, 0)
            wait_g(j + 1, 1)
            start_o(j + 1, 1)
            wait_o(j, 0)
            start_g(j + 2, 0)
            wait_o(j + 1, 1)
            start_g(j + 3, 1)
            return carry

        lax.fori_loop(0, npair - 1, pair, 0)
        j = nb - 2
        wait_g(j, 0)
        start_o(j, 0)
        wait_g(j + 1, 1)
        start_o(j + 1, 1)
        wait_o(j, 0)
        wait_o(j + 1, 1)

    def make(table, idx):
        return pl.kernel(
            body,
            out_type=jax.ShapeDtypeStruct((ep, d), jnp.int32),
            mesh=mesh,
            scratch_types=[
                pltpu.VMEM((per_tile,), jnp.int32),
                pltpu.VMEM((b, d), jnp.int32),
                pltpu.VMEM((b, d), jnp.int32),
                pltpu.SemaphoreType.DMA,
                pltpu.SemaphoreType.DMA,
                pltpu.SemaphoreType.DMA,
                pltpu.SemaphoreType.DMA,
            ],
        )(table, idx)

    return make


def _make_scatter(n, ep, h, c):
    ntab = ((n + _NTILE * 8) // (_NTILE * 8)) * (_NTILE * 8)
    br = 128
    per_tile_b = (ep // br) // _NTILE
    npair = per_tile_b // 2
    tab_slice = ntab // _NTILE
    rows_lo = (n // _NTILE) // 8 * 8
    rows_hi = n - rows_lo * (_NTILE - 1)
    vper = c // 16
    mesh = plsc.VectorSubcoreMesh(core_axis_name="c", subcore_axis_name="s")

    @functools.partial(
        pl.kernel,
        out_type=jax.ShapeDtypeStruct((h, n, c), jnp.float32),
        mesh=mesh,
        scratch_types=[
            pltpu.VMEM((128, c), jnp.float32),
            pltpu.VMEM((br, c), jnp.float32),
            pltpu.VMEM((br, c), jnp.float32),
            pltpu.VMEM((128,), jnp.int32),
            pltpu.VMEM((128,), jnp.int32),
            pltpu.VMEM_SHARED((ntab, c), jnp.float32),
            pltpu.SemaphoreType.DMA,
            pltpu.SemaphoreType.DMA,
        ],
    )
    def scat(msg_hbm, dstb_hbm, agg_hbm, zero_v, rows0, rows1,
             ia0, ia1, table, s0, s1):
        cid = lax.axis_index("c")
        sid = lax.axis_index("s")
        rows = (rows0, rows1)
        ia = (ia0, ia1)
        sem = (s0, s1)

        def zbody(i, carry):
            r = i // vper
            col = (i % vper) * 16
            zero_v[r, pl.ds(col, 16)] = jnp.zeros((16,), jnp.float32)
            return carry

        lax.fori_loop(0, 128 * vper, zbody, 0)
        tab_base = sid * tab_slice

        for ck in range(h // _NSC):
            chunk = cid * (h // _NSC) + ck
            off = 0
            left = tab_slice
            while left > 0:
                step = min(128, left)
                pltpu.sync_copy(zero_v.at[pl.ds(0, step)],
                                table.at[pl.ds(tab_base + off, step)])
                off += step
                left -= step
            plsc.subcore_barrier()

            def load(j, p):
                e0 = (sid * per_tile_b + j) * br
                pltpu.async_copy(dstb_hbm.at[pl.ds(e0, 128)], ia[p], sem[p])
                pltpu.async_copy(msg_hbm.at[chunk, pl.ds(e0, br)], rows[p],
                                 sem[p])

            def wait_load(j, p):
                e0 = (sid * per_tile_b + j) * br
                pltpu.make_async_copy(dstb_hbm.at[pl.ds(e0, 128)], ia[p],
                                      sem[p]).wait()
                pltpu.make_async_copy(msg_hbm.at[chunk, pl.ds(e0, br)],
                                      rows[p], sem[p]).wait()

            def add2(p):
                pltpu.sync_copy(rows[p], table.at[ia[p]], add=True)

            load(0, 0)

            def pair(j2, carry):
                j = 2 * j2
                load(j + 1, 1)
                wait_load(j, 0)
                add2(0)
                load(j + 2, 0)
                wait_load(j + 1, 1)
                add2(1)
                return carry

            lax.fori_loop(0, npair - 1, pair, 0)
            j = per_tile_b - 2
            load(j + 1, 1)
            wait_load(j, 0)
            add2(0)
            wait_load(j + 1, 1)
            add2(1)

            plsc.subcore_barrier()
            out_base = sid * rows_lo

            @pl.when(sid < _NTILE - 1)
            def _copy_lo():
                pltpu.sync_copy(table.at[pl.ds(out_base, rows_lo)],
                                agg_hbm.at[chunk, pl.ds(out_base, rows_lo)])

            @pl.when(sid == _NTILE - 1)
            def _copy_hi():
                base = rows_lo * (_NTILE - 1)
                pltpu.sync_copy(table.at[pl.ds(base, rows_hi)],
                                agg_hbm.at[chunk, pl.ds(base, rows_hi)])

            plsc.subcore_barrier()

    return scat


def _update_kernel(agg_ref, wcat_ref, bcat_ref, o_ref, stats_ref, *, h, c):
    i = pl.program_id(0)
    wcat = wcat_ref[...]
    o = bcat_ref[...]
    for hh in range(h):
        o = o + jnp.dot(agg_ref[hh, :, :], wcat[hh * c:(hh + 1) * c, :],
                        preferred_element_type=jnp.float32)
    o_ref[...] = o

    @pl.when(i == 0)
    def _init():
        stats_ref[...] = jnp.zeros_like(stats_ref)

    s = jnp.sum(o, axis=0, keepdims=True)
    ss = jnp.sum(o * o, axis=0, keepdims=True)
    stats_ref[0:1, :] = stats_ref[0:1, :] + s
    stats_ref[1:2, :] = stats_ref[1:2, :] + ss


def _pool_kernel(p_ref, stats_ref, g_ref, b_ref, batch_ref,
                 wfc_ref, bfc_ref, wout_ref, bout_ref, o_ref, *, n_nodes):
    p = p_ref[...]
    m = stats_ref[0:1, :] / n_nodes
    var = stats_ref[1:2, :] / n_nodes - m * m
    nf = _silu((p - m) * lax.rsqrt(var + _EPS) * g_ref[...] + b_ref[...])
    seg = batch_ref[...]
    gids = lax.broadcasted_iota(jnp.int32, (_G, 1), 0)
    onehot = (seg == gids).astype(jnp.float32)
    pooled = jnp.dot(onehot, nf, preferred_element_type=jnp.float32)
    counts = jnp.sum(onehot, axis=1, keepdims=True)
    pooled = pooled / jnp.maximum(counts, 1.0)
    feat = _silu(jnp.dot(pooled, wfc_ref[...],
                         preferred_element_type=jnp.float32) + bfc_ref[...])
    out = jnp.sum(feat * wout_ref[...], axis=1, keepdims=True) + bout_ref[...]
    o_ref[...] = out


def kernel(x, edge_attr, edge_index, batch, W_atom, b_atom, W_rbf1, b_rbf1,
           W_rbf2, b_rbf2, Wq, bq, Wk, bk, Wv, bv, We, Wcat, bcat, Wmu, bmu,
           Wml, bml, ln_msg_g, ln_msg_b, ln_a_g, ln_a_b, bn_g, bn_b,
           W_fc, b_fc, W_out, b_out):
    n, aif = x.shape
    e = edge_attr.shape[0]
    nfdim = W_atom.shape[1]
    efb = W_rbf1.shape[0]
    l_layers = Wq.shape[0]
    c = Wcat.shape[2]
    h = Wq.shape[2] // c
    hc = h * c

    ep = ((e + 4095) // 4096) * 4096
    pad = ep - e
    src = jnp.concatenate([edge_index[0], jnp.zeros((pad,), jnp.int32)])
    dst = jnp.concatenate([edge_index[1], jnp.zeros((pad,), jnp.int32)])
    dst_b = jnp.concatenate(
        [edge_index[1], jnp.full((pad,), n, jnp.int32)])
    edge_attr = jnp.concatenate(
        [edge_attr, jnp.zeros((pad, edge_attr.shape[1]), edge_attr.dtype)])

    f32 = jnp.float32
    row2 = lambda a: a.reshape(1, -1)

    nf0 = pl.pallas_call(
        _prologue_node_kernel,
        out_shape=jax.ShapeDtypeStruct((n, nfdim), f32),
        interpret=_IT,
    )(x, W_atom, row2(b_atom))

    eblk = _blk(ep, 5120)
    ef = pl.pallas_call(
        functools.partial(_prologue_edge_kernel, efb=efb),
        grid=(ep // eblk,),
        in_specs=[
            pl.BlockSpec((eblk, edge_attr.shape[1]), lambda i: (i, 0)),
            pl.BlockSpec((efb, nfdim), lambda i: (0, 0)),
            pl.BlockSpec((1, nfdim), lambda i: (0, 0)),
            pl.BlockSpec((nfdim, nfdim), lambda i: (0, 0)),
            pl.BlockSpec((1, nfdim), lambda i: (0, 0)),
        ],
        out_specs=pl.BlockSpec((eblk, nfdim), lambda i: (i, 0)),
        out_shape=jax.ShapeDtypeStruct((ep, nfdim), f32),
        interpret=_IT,
    )(edge_attr, W_rbf1, row2(b_rbf1), W_rbf2, row2(b_rbf2))

    nblk = _blk(n, 2000)
    ngrid = n // nblk

    prev = nf0
    stats = None
    for l in range(l_layers):
        i32 = jnp.int32
        nq = hc // c
        hp = nq // 2
        wspecs = [
            pl.BlockSpec((nfdim, hc), lambda i: (0, 0)),
            pl.BlockSpec((1, hc), lambda i: (0, 0)),
        ] * 3
        out_shapes = (jax.ShapeDtypeStruct((n, 3 * hp * 128), i32),
                      jax.ShapeDtypeStruct((n, nq * 128), i32))
        out_specs = (pl.BlockSpec((nblk, 3 * hp * 128), lambda i: (i, 0)),
                     pl.BlockSpec((nblk, nq * 128), lambda i: (i, 0)))
        wargs = (Wq[l], row2(bq[l]), Wk[l], row2(bk[l]), Wv[l], row2(bv[l]))
        if l == 0:
            qkv_t, kv_t = pl.pallas_call(
                functools.partial(_proj_kernel, hc=hc, c=c),
                grid=(ngrid,),
                in_specs=[pl.BlockSpec((nblk, nfdim), lambda i: (i, 0))] + wspecs,
                out_specs=out_specs,
                out_shape=out_shapes,
                interpret=_IT,
            )(prev, *wargs)
        else:
            qkv_t, kv_t = pl.pallas_call(
                functools.partial(_proj_bn_kernel, hc=hc, c=c, n_nodes=n),
                grid=(ngrid,),
                in_specs=[
                    pl.BlockSpec((nblk, nfdim), lambda i: (i, 0)),
                    pl.BlockSpec((2, nfdim), lambda i: (0, 0)),
                    pl.BlockSpec((1, nfdim), lambda i: (0, 0)),
                    pl.BlockSpec((1, nfdim), lambda i: (0, 0)),
                ] + wspecs,
                out_specs=out_specs,
                out_shape=out_shapes,
                interpret=_IT,
            )(prev, stats, row2(bn_g[l - 1]), row2(bn_b[l - 1]), *wargs)

        eblk2 = _blk(ep, 5120)
        e_t = pl.pallas_call(
            _eproj_kernel,
            grid=(ep // eblk2,),
            in_specs=[
                pl.BlockSpec((eblk2, nfdim), lambda i: (i, 0)),
                pl.BlockSpec((nfdim, hc), lambda i: (0, 0)),
            ],
            out_specs=pl.BlockSpec((eblk2, hc), lambda i: (i, 0)),
            out_shape=jax.ShapeDtypeStruct((ep, hc), f32),
            interpret=_IT,
        )(ef, We[l])

        g_qkv = _make_gather(3 * hp * 128, ep, 64)(qkv_t, dst)
        g_kv = _make_gather(nq * 128, ep, 64)(kv_t, src)

        eblk3 = _blk(ep, 1024)
        msg = pl.pallas_call(
            functools.partial(_edge_kernel, h=h, c=c),
            grid=(ep // eblk3,),
            in_specs=[
                pl.BlockSpec((eblk3, 3 * hp * 128), lambda i: (i, 0)),
                pl.BlockSpec((eblk3, nq * 128), lambda i: (i, 0)),
                pl.BlockSpec((eblk3, hc), lambda i: (i, 0)),
                pl.BlockSpec((3 * c, 3 * c), lambda i: (0, 0)),
                pl.BlockSpec((1, 3 * c), lambda i: (0, 0)),
                pl.BlockSpec((3 * c, c), lambda i: (0, 0)),
                pl.BlockSpec((1, c), lambda i: (0, 0)),
                pl.BlockSpec((1, 3 * c), lambda i: (0, 0)),
                pl.BlockSpec((1, 3 * c), lambda i: (0, 0)),
                pl.BlockSpec((1, c), lambda i: (0, 0)),
                pl.BlockSpec((1, c), lambda i: (0, 0)),
            ],
            out_specs=pl.BlockSpec((h, eblk3, c), lambda i: (0, i, 0)),
            out_shape=jax.ShapeDtypeStruct((h, ep, c), f32),
            interpret=_IT,
        )(g_qkv, g_kv, e_t, Wmu[l], row2(bmu[l]), Wml[l], row2(bml[l]),
          row2(ln_a_g[l]), row2(ln_a_b[l]), row2(ln_msg_g[l]),
          row2(ln_msg_b[l]))

        agg = _make_scatter(n, ep, h, c)(msg, dst_b)

        out_l, stats = pl.pallas_call(
            functools.partial(_update_kernel, h=h, c=c),
            grid=(ngrid,),
            in_specs=[
                pl.BlockSpec((h, nblk, c), lambda i: (0, i, 0)),
                pl.BlockSpec((hc, nfdim), lambda i: (0, 0)),
                pl.BlockSpec((1, nfdim), lambda i: (0, 0)),
            ],
            out_specs=(pl.BlockSpec((nblk, nfdim), lambda i: (i, 0)),
                       pl.BlockSpec((2, nfdim), lambda i: (0, 0))),
            out_shape=(jax.ShapeDtypeStruct((n, nfdim), f32),
                       jax.ShapeDtypeStruct((2, nfdim), f32)),
            interpret=_IT,
        )(agg, Wcat[l], row2(bcat[l]))
        prev = out_l

    out = pl.pallas_call(
        functools.partial(_pool_kernel, n_nodes=n),
        out_shape=jax.ShapeDtypeStruct((_G, 1), f32),
        interpret=_IT,
    )(prev, stats, row2(bn_g[l_layers - 1]), row2(bn_b[l_layers - 1]),
      batch.reshape(1, n), W_fc, row2(b_fc), W_out.reshape(1, -1),
      row2(b_out))
    return out.reshape(_G)


# speedup vs baseline: 4.0360x; 1.0944x over previous
import functools
import math

import jax
import jax.numpy as jnp
from jax import lax
from jax.experimental import pallas as pl
from jax.experimental.pallas import tpu as pltpu
from jax.experimental.pallas import tpu_sc as plsc

_IT = False
_G = 256
_EPS = 1e-5

_NSC = 2
_NTILE = 16


def _sigmoid(x):
    return 1.0 / (1.0 + jnp.exp(-x))


def _silu(x):
    return x * _sigmoid(x)


def _blk(n, target):
    if n % target == 0:
        return target
    return n


def _prologue_node_kernel(x_ref, w_ref, b_ref, o_ref):
    o_ref[...] = jnp.dot(x_ref[...], w_ref[...],
                         preferred_element_type=jnp.float32) + b_ref[...]


def _prologue_edge_kernel(ea_ref, w1_ref, b1_ref, w2_ref, b2_ref, o_ref, *,
                          efb):
    ea = ea_ref[...]
    d = jnp.sqrt(jnp.sum(ea * ea, axis=1, keepdims=True))
    step = 8.0 / (efb - 1)
    centers = lax.broadcasted_iota(jnp.int32, (1, efb), 1).astype(jnp.float32) * step
    gamma = 1.0 / (step * step)
    diff = d - centers
    rbf = jnp.exp(-gamma * diff * diff)
    z = jnp.dot(rbf, w1_ref[...], preferred_element_type=jnp.float32) + b1_ref[...]
    sp = jnp.maximum(z, 0.0) + jnp.log(1.0 + jnp.exp(-jnp.abs(z)))
    o_ref[...] = jnp.dot(sp, w2_ref[...],
                         preferred_element_type=jnp.float32) + b2_ref[...]


def _rne16(u):
    return u + jnp.uint32(0x7FFF) + ((u >> 16) & jnp.uint32(1))


def _pack2(lo, hi):
    ul = _rne16(lax.bitcast_convert_type(lo, jnp.uint32))
    uh = _rne16(lax.bitcast_convert_type(hi, jnp.uint32))
    packed = (uh & jnp.uint32(0xFFFF0000)) | (ul >> 16)
    return lax.bitcast_convert_type(packed, jnp.int32)


def _unpack(x, hi):
    u = lax.bitcast_convert_type(x, jnp.uint32)
    v = (u & jnp.uint32(0xFFFF0000)) if hi else (u << 16)
    return lax.bitcast_convert_type(v, jnp.float32)


def _store_tables(q, k, v, qkv_ref, kv_ref, c):
    nq = q.shape[1] // c
    hp = nq // 2
    for t in range(hp):
        s0 = slice((2 * t) * c, (2 * t + 1) * c)
        s1 = slice((2 * t + 1) * c, (2 * t + 2) * c)
        d = slice(t * c, (t + 1) * c)
        d2 = slice((hp + t) * c, (hp + t + 1) * c)
        d3 = slice((2 * hp + t) * c, (2 * hp + t + 1) * c)
        kp = _pack2(k[:, s0], k[:, s1])
        vp = _pack2(v[:, s0], v[:, s1])
        qkv_ref[:, d] = _pack2(q[:, s0], q[:, s1])
        qkv_ref[:, d2] = kp
        qkv_ref[:, d3] = vp
        kv_ref[:, d] = kp
        kv_ref[:, d2] = vp


def _proj_kernel(p_ref, wq_ref, bq_ref, wk_ref, bk_ref, wv_ref, bv_ref,
                 qkv_ref, kv_ref, *, hc, c):
    nf = p_ref[...]
    q = jnp.dot(nf, wq_ref[...], preferred_element_type=jnp.float32) + bq_ref[...]
    k = jnp.dot(nf, wk_ref[...], preferred_element_type=jnp.float32) + bk_ref[...]
    v = jnp.dot(nf, wv_ref[...], preferred_element_type=jnp.float32) + bv_ref[...]
    _store_tables(q, k, v, qkv_ref, kv_ref, c)


def _proj_bn_kernel(p_ref, stats_ref, g_ref, b_ref,
                    wq_ref, bq_ref, wk_ref, bk_ref, wv_ref, bv_ref,
                    qkv_ref, kv_ref, *, hc, c, n_nodes):
    p = p_ref[...]
    m = stats_ref[0:1, :] / n_nodes
    var = stats_ref[1:2, :] / n_nodes - m * m
    xb = (p - m) * lax.rsqrt(var + _EPS) * g_ref[...] + b_ref[...]
    nf = _silu(xb)
    q = jnp.dot(nf, wq_ref[...], preferred_element_type=jnp.float32) + bq_ref[...]
    k = jnp.dot(nf, wk_ref[...], preferred_element_type=jnp.float32) + bk_ref[...]
    v = jnp.dot(nf, wv_ref[...], preferred_element_type=jnp.float32) + bv_ref[...]
    _store_tables(q, k, v, qkv_ref, kv_ref, c)


def _eproj_kernel(ef_ref, we_ref, o_ref):
    o_ref[...] = jnp.dot(ef_ref[...], we_ref[...],
                         preferred_element_type=jnp.float32)


def _edge_kernel(gqkv_ref, gkv_ref, e_ref, wmu_ref, bmu_ref,
                 wml_ref, bml_ref,
                 lag_ref, lab_ref, lmg_ref, lmb_ref, o_ref, *, h, c):
    scale = 1.0 / math.sqrt(3.0 * c)
    f32 = jnp.float32
    wmu = wmu_ref[...]
    bmu = bmu_ref[...]
    wml = wml_ref[...]
    bml = bml_ref[...]
    hp = h // 2
    for hh in range(h):
        sl = slice(hh * c, (hh + 1) * c)
        t, odd = hh // 2, hh % 2
        ts = slice(t * c, (t + 1) * c)
        t2 = slice((hp + t) * c, (hp + t + 1) * c)
        t3 = slice((2 * hp + t) * c, (2 * hp + t + 1) * c)
        q = _unpack(gqkv_ref[:, ts], odd)
        k_i = _unpack(gqkv_ref[:, t2], odd)
        v_i = _unpack(gqkv_ref[:, t3], odd)
        k_j = _unpack(gkv_ref[:, ts], odd)
        v_j = _unpack(gkv_ref[:, t2], odd)
        e = e_ref[:, sl]
        a1 = q * k_i * scale
        a2 = q * k_j * scale
        a3 = q * e * scale
        s = (jnp.sum(a1, axis=1, keepdims=True)
             + jnp.sum(a2, axis=1, keepdims=True)
             + jnp.sum(a3, axis=1, keepdims=True))
        ss = (jnp.sum(a1 * a1, axis=1, keepdims=True)
              + jnp.sum(a2 * a2, axis=1, keepdims=True)
              + jnp.sum(a3 * a3, axis=1, keepdims=True))
        m = s / (3.0 * c)
        var = ss / (3.0 * c) - m * m
        r = lax.rsqrt(var + _EPS)
        g1 = _sigmoid((a1 - m) * r * lag_ref[:, 0:c] + lab_ref[:, 0:c])
        g2 = _sigmoid((a2 - m) * r * lag_ref[:, c:2 * c] + lab_ref[:, c:2 * c])
        g3 = _sigmoid((a3 - m) * r * lag_ref[:, 2 * c:3 * c] + lab_ref[:, 2 * c:3 * c])
        m2 = (jnp.dot(v_i, wmu[0:c, :], preferred_element_type=jnp.float32)
              + jnp.dot(v_j, wmu[c:2 * c, :], preferred_element_type=jnp.float32)
              + jnp.dot(e, wmu[2 * c:3 * c, :], preferred_element_type=jnp.float32)
              + bmu)
        m3 = (jnp.dot(m2[:, 0:c] * g1, wml[0:c, :],
                      preferred_element_type=jnp.float32)
              + jnp.dot(m2[:, c:2 * c] * g2, wml[c:2 * c, :],
                        preferred_element_type=jnp.float32)
              + jnp.dot(m2[:, 2 * c:3 * c] * g3, wml[2 * c:3 * c, :],
                        preferred_element_type=jnp.float32)
              + bml)
        mm = jnp.mean(m3, axis=1, keepdims=True)
        mv = jnp.mean(m3 * m3, axis=1, keepdims=True) - mm * mm
        msg = (m3 - mm) * lax.rsqrt(mv + _EPS) * lmg_ref[...] + lmb_ref[...]
        o_ref[hh, :, :] = msg


def _make_gather(d, ep, b):
    per_tile = ep // (_NSC * _NTILE)
    nb = per_tile // b
    npair = nb // 2
    mesh = plsc.VectorSubcoreMesh(core_axis_name="c", subcore_axis_name="s")

    def body(table_hbm, idx_hbm, out_hbm, idx_v, r0, r1, g0, g1, o0, o1):
        cid = lax.axis_index("c")
        sid = lax.axis_index("s")
        wid = sid * _NSC + cid
        base = wid * per_tile
        pltpu.sync_copy(idx_hbm.at[pl.ds(base, per_tile)], idx_v)
        bufs = (r0, r1)
        gsem = (g0, g1)
        osem = (o0, o1)

        def start_g(j, p):
            pltpu.async_copy(table_hbm.at[idx_v.at[pl.ds(j * b, b)]],
                             bufs[p], gsem[p])

        def wait_g(j, p):
            pltpu.make_async_copy(table_hbm.at[idx_v.at[pl.ds(j * b, b)]],
                                  bufs[p], gsem[p]).wait()

        def start_o(j, p):
            pltpu.async_copy(bufs[p], out_hbm.at[pl.ds(base + j * b, b)],
                             osem[p])

        def wait_o(j, p):
            pltpu.make_async_copy(bufs[p],
                                  out_hbm.at[pl.ds(base + j * b, b)],
                                  osem[p]).wait()

        start_g(0, 0)
        start_g(1, 1)

        def pair(j2, carry):
            j = 2 * j2
            wait_g(j, 0)
            start_o(j, 0)
            wait_g(j + 1, 1)
            start_o(j + 1, 1)
            wait_o(j, 0)
            start_g(j + 2, 0)
            wait_o(j + 1, 1)
            start_g(j + 3, 1)
            return carry

        lax.fori_loop(0, npair - 1, pair, 0)
        j = nb - 2
        wait_g(j, 0)
        start_o(j, 0)
        wait_g(j + 1, 1)
        start_o(j + 1, 1)
        wait_o(j, 0)
        wait_o(j + 1, 1)

    def make(table, idx):
        return pl.kernel(
            body,
            out_type=jax.ShapeDtypeStruct((ep, d), jnp.int32),
            mesh=mesh,
            scratch_types=[
                pltpu.VMEM((per_tile,), jnp.int32),
                pltpu.VMEM((b, d), jnp.int32),
                pltpu.VMEM((b, d), jnp.int32),
                pltpu.SemaphoreType.DMA,
                pltpu.SemaphoreType.DMA,
                pltpu.SemaphoreType.DMA,
                pltpu.SemaphoreType.DMA,
            ],
        )(table, idx)

    return make


def _make_gather2(d1, d2, ep, b):
    per_tile = ep // (_NSC * _NTILE)
    nb = per_tile // b
    npair = nb // 2
    mesh = plsc.VectorSubcoreMesh(core_axis_name="c", subcore_axis_name="s")

    def body(tab1, idx1_h, tab2, idx2_h, out1, out2, ix1, ix2,
             a0, a1, b0, b1, ga0, ga1, gb0, gb1, oa0, oa1, ob0, ob1):
        cid = lax.axis_index("c")
        sid = lax.axis_index("s")
        wid = sid * _NSC + cid
        base = wid * per_tile
        pltpu.sync_copy(idx1_h.at[pl.ds(base, per_tile)], ix1)
        pltpu.sync_copy(idx2_h.at[pl.ds(base, per_tile)], ix2)
        abufs = (a0, a1)
        bbufs = (b0, b1)
        gas = (ga0, ga1)
        gbs = (gb0, gb1)
        oas = (oa0, oa1)
        obs = (ob0, ob1)

        def start_g(j, p):
            pltpu.async_copy(tab1.at[ix1.at[pl.ds(j * b, b)]],
                             abufs[p], gas[p])
            pltpu.async_copy(tab2.at[ix2.at[pl.ds(j * b, b)]],
                             bbufs[p], gbs[p])

        def wait_g(j, p):
            pltpu.make_async_copy(tab1.at[ix1.at[pl.ds(j * b, b)]],
                                  abufs[p], gas[p]).wait()
            pltpu.make_async_copy(tab2.at[ix2.at[pl.ds(j * b, b)]],
                                  bbufs[p], gbs[p]).wait()

        def start_o(j, p):
            pltpu.async_copy(abufs[p], out1.at[pl.ds(base + j * b, b)],
                             oas[p])
            pltpu.async_copy(bbufs[p], out2.at[pl.ds(base + j * b, b)],
                             obs[p])

        def wait_o(j, p):
            pltpu.make_async_copy(abufs[p],
                                  out1.at[pl.ds(base + j * b, b)],
                                  oas[p]).wait()
            pltpu.make_async_copy(bbufs[p],
                                  out2.at[pl.ds(base + j * b, b)],
                                  obs[p]).wait()

        start_g(0, 0)
        start_g(1, 1)

        def pair(j2, carry):
            j = 2 * j2
            wait_g(j, 0)
            start_o(j, 0)
            wait_g(j + 1, 1)
            start_o(j + 1, 1)
            wait_o(j, 0)
            start_g(j + 2, 0)
            wait_o(j + 1, 1)
            start_g(j + 3, 1)
            return carry

        lax.fori_loop(0, npair - 1, pair, 0)
        j = nb - 2
        wait_g(j, 0)
        start_o(j, 0)
        wait_g(j + 1, 1)
        start_o(j + 1, 1)
        wait_o(j, 0)
        wait_o(j + 1, 1)

    def make(tab1, idx1, tab2, idx2):
        return pl.kernel(
            body,
            out_type=(jax.ShapeDtypeStruct((ep, d1), jnp.int32),
                      jax.ShapeDtypeStruct((ep, d2), jnp.int32)),
            mesh=mesh,
            scratch_types=[
                pltpu.VMEM((per_tile,), jnp.int32),
                pltpu.VMEM((per_tile,), jnp.int32),
                pltpu.VMEM((b, d1), jnp.int32),
                pltpu.VMEM((b, d1), jnp.int32),
                pltpu.VMEM((b, d2), jnp.int32),
                pltpu.VMEM((b, d2), jnp.int32),
                pltpu.SemaphoreType.DMA,
                pltpu.SemaphoreType.DMA,
                pltpu.SemaphoreType.DMA,
                pltpu.SemaphoreType.DMA,
                pltpu.SemaphoreType.DMA,
                pltpu.SemaphoreType.DMA,
                pltpu.SemaphoreType.DMA,
                pltpu.SemaphoreType.DMA,
            ],
        )(tab1, idx1, tab2, idx2)

    return make


def _make_scatter(n, ep, h, c):
    ntab = ((n + _NTILE * 8) // (_NTILE * 8)) * (_NTILE * 8)
    br = 128
    per_tile_b = (ep // br) // _NTILE
    npair = per_tile_b // 2
    tab_slice = ntab // _NTILE
    rows_lo = (n // _NTILE) // 8 * 8
    rows_hi = n - rows_lo * (_NTILE - 1)
    vper = c // 16
    mesh = plsc.VectorSubcoreMesh(core_axis_name="c", subcore_axis_name="s")

    @functools.partial(
        pl.kernel,
        out_type=jax.ShapeDtypeStruct((h, n, c), jnp.float32),
        mesh=mesh,
        scratch_types=[
            pltpu.VMEM((128, c), jnp.float32),
            pltpu.VMEM((br, c), jnp.float32),
            pltpu.VMEM((br, c), jnp.float32),
            pltpu.VMEM((128,), jnp.int32),
            pltpu.VMEM((128,), jnp.int32),
            pltpu.VMEM_SHARED((ntab, c), jnp.float32),
            pltpu.SemaphoreType.DMA,
            pltpu.SemaphoreType.DMA,
        ],
    )
    def scat(msg_hbm, dstb_hbm, agg_hbm, zero_v, rows0, rows1,
             ia0, ia1, table, s0, s1):
        cid = lax.axis_index("c")
        sid = lax.axis_index("s")
        rows = (rows0, rows1)
        ia = (ia0, ia1)
        sem = (s0, s1)

        def zbody(i, carry):
            r = i // vper
            col = (i % vper) * 16
            zero_v[r, pl.ds(col, 16)] = jnp.zeros((16,), jnp.float32)
            return carry

        lax.fori_loop(0, 128 * vper, zbody, 0)
        tab_base = sid * tab_slice

        for ck in range(h // _NSC):
            chunk = cid * (h // _NSC) + ck
            off = 0
            left = tab_slice
            while left > 0:
                step = min(128, left)
                pltpu.sync_copy(zero_v.at[pl.ds(0, step)],
                                table.at[pl.ds(tab_base + off, step)])
                off += step
                left -= step
            plsc.subcore_barrier()

            def load(j, p):
                e0 = (sid * per_tile_b + j) * br
                pltpu.async_copy(dstb_hbm.at[pl.ds(e0, 128)], ia[p], sem[p])
                pltpu.async_copy(msg_hbm.at[chunk, pl.ds(e0, br)], rows[p],
                                 sem[p])

            def wait_load(j, p):
                e0 = (sid * per_tile_b + j) * br
                pltpu.make_async_copy(dstb_hbm.at[pl.ds(e0, 128)], ia[p],
                                      sem[p]).wait()
                pltpu.make_async_copy(msg_hbm.at[chunk, pl.ds(e0, br)],
                                      rows[p], sem[p]).wait()

            def add2(p):
                pltpu.sync_copy(rows[p], table.at[ia[p]], add=True)

            load(0, 0)

            def pair(j2, carry):
                j = 2 * j2
                load(j + 1, 1)
                wait_load(j, 0)
                add2(0)
                load(j + 2, 0)
                wait_load(j + 1, 1)
                add2(1)
                return carry

            lax.fori_loop(0, npair - 1, pair, 0)
            j = per_tile_b - 2
            load(j + 1, 1)
            wait_load(j, 0)
            add2(0)
            wait_load(j + 1, 1)
            add2(1)

            plsc.subcore_barrier()
            out_base = sid * rows_lo

            @pl.when(sid < _NTILE - 1)
            def _copy_lo():
                pltpu.sync_copy(table.at[pl.ds(out_base, rows_lo)],
                                agg_hbm.at[chunk, pl.ds(out_base, rows_lo)])

            @pl.when(sid == _NTILE - 1)
            def _copy_hi():
                base = rows_lo * (_NTILE - 1)
                pltpu.sync_copy(table.at[pl.ds(base, rows_hi)],
                                agg_hbm.at[chunk, pl.ds(base, rows_hi)])

            plsc.subcore_barrier()

    return scat


def _update_kernel(agg_ref, wcat_ref, bcat_ref, o_ref, stats_ref, *, h, c):
    i = pl.program_id(0)
    wcat = wcat_ref[...]
    o = bcat_ref[...]
    for hh in range(h):
        o = o + jnp.dot(agg_ref[hh, :, :], wcat[hh * c:(hh + 1) * c, :],
                        preferred_element_type=jnp.float32)
    o_ref[...] = o

    @pl.when(i == 0)
    def _init():
        stats_ref[...] = jnp.zeros_like(stats_ref)

    s = jnp.sum(o, axis=0, keepdims=True)
    ss = jnp.sum(o * o, axis=0, keepdims=True)
    stats_ref[0:1, :] = stats_ref[0:1, :] + s
    stats_ref[1:2, :] = stats_ref[1:2, :] + ss


def _pool_kernel(p_ref, stats_ref, g_ref, b_ref, batch_ref,
                 wfc_ref, bfc_ref, wout_ref, bout_ref, o_ref, *, n_nodes):
    p = p_ref[...]
    m = stats_ref[0:1, :] / n_nodes
    var = stats_ref[1:2, :] / n_nodes - m * m
    nf = _silu((p - m) * lax.rsqrt(var + _EPS) * g_ref[...] + b_ref[...])
    seg = batch_ref[...]
    gids = lax.broadcasted_iota(jnp.int32, (_G, 1), 0)
    onehot = (seg == gids).astype(jnp.float32)
    pooled = jnp.dot(onehot, nf, preferred_element_type=jnp.float32)
    counts = jnp.sum(onehot, axis=1, keepdims=True)
    pooled = pooled / jnp.maximum(counts, 1.0)
    feat = _silu(jnp.dot(pooled, wfc_ref[...],
                         preferred_element_type=jnp.float32) + bfc_ref[...])
    out = jnp.sum(feat * wout_ref[...], axis=1, keepdims=True) + bout_ref[...]
    o_ref[...] = out


def kernel(x, edge_attr, edge_index, batch, W_atom, b_atom, W_rbf1, b_rbf1,
           W_rbf2, b_rbf2, Wq, bq, Wk, bk, Wv, bv, We, Wcat, bcat, Wmu, bmu,
           Wml, bml, ln_msg_g, ln_msg_b, ln_a_g, ln_a_b, bn_g, bn_b,
           W_fc, b_fc, W_out, b_out):
    n, aif = x.shape
    e = edge_attr.shape[0]
    nfdim = W_atom.shape[1]
    efb = W_rbf1.shape[0]
    l_layers = Wq.shape[0]
    c = Wcat.shape[2]
    h = Wq.shape[2] // c
    hc = h * c

    ep = ((e + 4095) // 4096) * 4096
    pad = ep - e
    src = jnp.concatenate([edge_index[0], jnp.zeros((pad,), jnp.int32)])
    dst = jnp.concatenate([edge_index[1], jnp.zeros((pad,), jnp.int32)])
    dst_b = jnp.concatenate(
        [edge_index[1], jnp.full((pad,), n, jnp.int32)])
    edge_attr = jnp.concatenate(
        [edge_attr, jnp.zeros((pad, edge_attr.shape[1]), edge_attr.dtype)])

    f32 = jnp.float32
    row2 = lambda a: a.reshape(1, -1)

    nf0 = pl.pallas_call(
        _prologue_node_kernel,
        out_shape=jax.ShapeDtypeStruct((n, nfdim), f32),
        interpret=_IT,
    )(x, W_atom, row2(b_atom))

    eblk = _blk(ep, 5120)
    ef = pl.pallas_call(
        functools.partial(_prologue_edge_kernel, efb=efb),
        grid=(ep // eblk,),
        in_specs=[
            pl.BlockSpec((eblk, edge_attr.shape[1]), lambda i: (i, 0)),
            pl.BlockSpec((efb, nfdim), lambda i: (0, 0)),
            pl.BlockSpec((1, nfdim), lambda i: (0, 0)),
            pl.BlockSpec((nfdim, nfdim), lambda i: (0, 0)),
            pl.BlockSpec((1, nfdim), lambda i: (0, 0)),
        ],
        out_specs=pl.BlockSpec((eblk, nfdim), lambda i: (i, 0)),
        out_shape=jax.ShapeDtypeStruct((ep, nfdim), f32),
        interpret=_IT,
    )(edge_attr, W_rbf1, row2(b_rbf1), W_rbf2, row2(b_rbf2))

    nblk = _blk(n, 2000)
    ngrid = n // nblk

    prev = nf0
    stats = None
    for l in range(l_layers):
        i32 = jnp.int32
        nq = hc // c
        hp = nq // 2
        wspecs = [
            pl.BlockSpec((nfdim, hc), lambda i: (0, 0)),
            pl.BlockSpec((1, hc), lambda i: (0, 0)),
        ] * 3
        out_shapes = (jax.ShapeDtypeStruct((n, 3 * hp * 128), i32),
                      jax.ShapeDtypeStruct((n, nq * 128), i32))
        out_specs = (pl.BlockSpec((nblk, 3 * hp * 128), lambda i: (i, 0)),
                     pl.BlockSpec((nblk, nq * 128), lambda i: (i, 0)))
        wargs = (Wq[l], row2(bq[l]), Wk[l], row2(bk[l]), Wv[l], row2(bv[l]))
        if l == 0:
            qkv_t, kv_t = pl.pallas_call(
                functools.partial(_proj_kernel, hc=hc, c=c),
                grid=(ngrid,),
                in_specs=[pl.BlockSpec((nblk, nfdim), lambda i: (i, 0))] + wspecs,
                out_specs=out_specs,
                out_shape=out_shapes,
                interpret=_IT,
            )(prev, *wargs)
        else:
            qkv_t, kv_t = pl.pallas_call(
                functools.partial(_proj_bn_kernel, hc=hc, c=c, n_nodes=n),
                grid=(ngrid,),
                in_specs=[
                    pl.BlockSpec((nblk, nfdim), lambda i: (i, 0)),
                    pl.BlockSpec((2, nfdim), lambda i: (0, 0)),
                    pl.BlockSpec((1, nfdim), lambda i: (0, 0)),
                    pl.BlockSpec((1, nfdim), lambda i: (0, 0)),
                ] + wspecs,
                out_specs=out_specs,
                out_shape=out_shapes,
                interpret=_IT,
            )(prev, stats, row2(bn_g[l - 1]), row2(bn_b[l - 1]), *wargs)

        eblk2 = _blk(ep, 5120)
        e_t = pl.pallas_call(
            _eproj_kernel,
            grid=(ep // eblk2,),
            in_specs=[
                pl.BlockSpec((eblk2, nfdim), lambda i: (i, 0)),
                pl.BlockSpec((nfdim, hc), lambda i: (0, 0)),
            ],
            out_specs=pl.BlockSpec((eblk2, hc), lambda i: (i, 0)),
            out_shape=jax.ShapeDtypeStruct((ep, hc), f32),
            interpret=_IT,
        )(ef, We[l])

        g_qkv, g_kv = _make_gather2(3 * hp * 128, nq * 128, ep, 32)(
            qkv_t, dst, kv_t, src)

        eblk3 = _blk(ep, 1024)
        msg = pl.pallas_call(
            functools.partial(_edge_kernel, h=h, c=c),
            grid=(ep // eblk3,),
            in_specs=[
                pl.BlockSpec((eblk3, 3 * hp * 128), lambda i: (i, 0)),
                pl.BlockSpec((eblk3, nq * 128), lambda i: (i, 0)),
                pl.BlockSpec((eblk3, hc), lambda i: (i, 0)),
                pl.BlockSpec((3 * c, 3 * c), lambda i: (0, 0)),
                pl.BlockSpec((1, 3 * c), lambda i: (0, 0)),
                pl.BlockSpec((3 * c, c), lambda i: (0, 0)),
                pl.BlockSpec((1, c), lambda i: (0, 0)),
                pl.BlockSpec((1, 3 * c), lambda i: (0, 0)),
                pl.BlockSpec((1, 3 * c), lambda i: (0, 0)),
                pl.BlockSpec((1, c), lambda i: (0, 0)),
                pl.BlockSpec((1, c), lambda i: (0, 0)),
            ],
            out_specs=pl.BlockSpec((h, eblk3, c), lambda i: (0, i, 0)),
            out_shape=jax.ShapeDtypeStruct((h, ep, c), f32),
            interpret=_IT,
        )(g_qkv, g_kv, e_t, Wmu[l], row2(bmu[l]), Wml[l], row2(bml[l]),
          row2(ln_a_g[l]), row2(ln_a_b[l]), row2(ln_msg_g[l]),
          row2(ln_msg_b[l]))

        agg = _make_scatter(n, ep, h, c)(msg, dst_b)

        out_l, stats = pl.pallas_call(
            functools.partial(_update_kernel, h=h, c=c),
            grid=(ngrid,),
            in_specs=[
                pl.BlockSpec((h, nblk, c), lambda i: (0, i, 0)),
                pl.BlockSpec((hc, nfdim), lambda i: (0, 0)),
                pl.BlockSpec((1, nfdim), lambda i: (0, 0)),
            ],
            out_specs=(pl.BlockSpec((nblk, nfdim), lambda i: (i, 0)),
                       pl.BlockSpec((2, nfdim), lambda i: (0, 0))),
            out_shape=(jax.ShapeDtypeStruct((n, nfdim), f32),
                       jax.ShapeDtypeStruct((2, nfdim), f32)),
            interpret=_IT,
        )(agg, Wcat[l], row2(bcat[l]))
        prev = out_l

    out = pl.pallas_call(
        functools.partial(_pool_kernel, n_nodes=n),
        out_shape=jax.ShapeDtypeStruct((_G, 1), f32),
        interpret=_IT,
    )(prev, stats, row2(bn_g[l_layers - 1]), row2(bn_b[l_layers - 1]),
      batch.reshape(1, n), W_fc, row2(b_fc), W_out.reshape(1, -1),
      row2(b_out))
    return out.reshape(_G)


# speedup vs baseline: 4.0367x; 1.0002x over previous
import functools
import math

import jax
import jax.numpy as jnp
from jax import lax
from jax.experimental import pallas as pl
from jax.experimental.pallas import tpu as pltpu
from jax.experimental.pallas import tpu_sc as plsc

_G = 256
_EPS = 1e-5

_NSC = 2
_NTILE = 16


def _sigmoid(x):
    return 1.0 / (1.0 + jnp.exp(-x))


def _silu(x):
    return x * _sigmoid(x)


def _blk(n, target):
    if n % target == 0:
        return target
    return n


def _prologue_node_kernel(x_ref, w_ref, b_ref, o_ref):
    o_ref[...] = jnp.dot(x_ref[...], w_ref[...],
                         preferred_element_type=jnp.float32) + b_ref[...]


def _prologue_edge_kernel(ea_ref, w1_ref, b1_ref, w2_ref, b2_ref, o_ref, *,
                          efb):
    ea = ea_ref[...]
    d = jnp.sqrt(jnp.sum(ea * ea, axis=1, keepdims=True))
    step = 8.0 / (efb - 1)
    centers = lax.broadcasted_iota(jnp.int32, (1, efb), 1).astype(jnp.float32) * step
    gamma = 1.0 / (step * step)
    diff = d - centers
    rbf = jnp.exp(-gamma * diff * diff)
    z = jnp.dot(rbf, w1_ref[...], preferred_element_type=jnp.float32) + b1_ref[...]
    sp = jnp.maximum(z, 0.0) + jnp.log(1.0 + jnp.exp(-jnp.abs(z)))
    o_ref[...] = jnp.dot(sp, w2_ref[...],
                         preferred_element_type=jnp.float32) + b2_ref[...]


def _rne16(u):
    return u + jnp.uint32(0x7FFF) + ((u >> 16) & jnp.uint32(1))


def _pack2(lo, hi):
    ul = _rne16(lax.bitcast_convert_type(lo, jnp.uint32))
    uh = _rne16(lax.bitcast_convert_type(hi, jnp.uint32))
    packed = (uh & jnp.uint32(0xFFFF0000)) | (ul >> 16)
    return lax.bitcast_convert_type(packed, jnp.int32)


def _unpack(x, hi):
    u = lax.bitcast_convert_type(x, jnp.uint32)
    v = (u & jnp.uint32(0xFFFF0000)) if hi else (u << 16)
    return lax.bitcast_convert_type(v, jnp.float32)


def _store_tables(q, k, v, qkv_ref, kv_ref, c):
    nq = q.shape[1] // c
    hp = nq // 2
    for t in range(hp):
        s0 = slice((2 * t) * c, (2 * t + 1) * c)
        s1 = slice((2 * t + 1) * c, (2 * t + 2) * c)
        d = slice(t * c, (t + 1) * c)
        d2 = slice((hp + t) * c, (hp + t + 1) * c)
        d3 = slice((2 * hp + t) * c, (2 * hp + t + 1) * c)
        kp = _pack2(k[:, s0], k[:, s1])
        vp = _pack2(v[:, s0], v[:, s1])
        qkv_ref[:, d] = _pack2(q[:, s0], q[:, s1])
        qkv_ref[:, d2] = kp
        qkv_ref[:, d3] = vp
        kv_ref[:, d] = kp
        kv_ref[:, d2] = vp


def _proj_kernel(p_ref, wq_ref, bq_ref, wk_ref, bk_ref, wv_ref, bv_ref,
                 qkv_ref, kv_ref, *, hc, c):
    nf = p_ref[...]
    q = jnp.dot(nf, wq_ref[...], preferred_element_type=jnp.float32) + bq_ref[...]
    k = jnp.dot(nf, wk_ref[...], preferred_element_type=jnp.float32) + bk_ref[...]
    v = jnp.dot(nf, wv_ref[...], preferred_element_type=jnp.float32) + bv_ref[...]
    _store_tables(q, k, v, qkv_ref, kv_ref, c)


def _proj_bn_kernel(p_ref, stats_ref, g_ref, b_ref,
                    wq_ref, bq_ref, wk_ref, bk_ref, wv_ref, bv_ref,
                    qkv_ref, kv_ref, *, hc, c, n_nodes):
    p = p_ref[...]
    m = stats_ref[0:1, :] / n_nodes
    var = stats_ref[1:2, :] / n_nodes - m * m
    xb = (p - m) * lax.rsqrt(var + _EPS) * g_ref[...] + b_ref[...]
    nf = _silu(xb)
    q = jnp.dot(nf, wq_ref[...], preferred_element_type=jnp.float32) + bq_ref[...]
    k = jnp.dot(nf, wk_ref[...], preferred_element_type=jnp.float32) + bk_ref[...]
    v = jnp.dot(nf, wv_ref[...], preferred_element_type=jnp.float32) + bv_ref[...]
    _store_tables(q, k, v, qkv_ref, kv_ref, c)


def _eproj_kernel(ef_ref, we_ref, o_ref):
    o_ref[...] = jnp.dot(ef_ref[...], we_ref[...],
                         preferred_element_type=jnp.float32)


def _edge_kernel(gqkv_ref, gkv_ref, e_ref, wmu_ref, bmu_ref,
                 wml_ref, bml_ref,
                 lag_ref, lab_ref, lmg_ref, lmb_ref, o_ref, *, h, c):
    scale = 1.0 / math.sqrt(3.0 * c)
    f32 = jnp.float32
    wmu = wmu_ref[...]
    bmu = bmu_ref[...]
    wml = wml_ref[...]
    bml = bml_ref[...]
    hp = h // 2
    for hh in range(h):
        sl = slice(hh * c, (hh + 1) * c)
        t, odd = hh // 2, hh % 2
        ts = slice(t * c, (t + 1) * c)
        t2 = slice((hp + t) * c, (hp + t + 1) * c)
        t3 = slice((2 * hp + t) * c, (2 * hp + t + 1) * c)
        q = _unpack(gqkv_ref[:, ts], odd)
        k_i = _unpack(gqkv_ref[:, t2], odd)
        v_i = _unpack(gqkv_ref[:, t3], odd)
        k_j = _unpack(gkv_ref[:, ts], odd)
        v_j = _unpack(gkv_ref[:, t2], odd)
        e = e_ref[:, sl]
        a1 = q * k_i * scale
        a2 = q * k_j * scale
        a3 = q * e * scale
        s = (jnp.sum(a1, axis=1, keepdims=True)
             + jnp.sum(a2, axis=1, keepdims=True)
             + jnp.sum(a3, axis=1, keepdims=True))
        ss = (jnp.sum(a1 * a1, axis=1, keepdims=True)
              + jnp.sum(a2 * a2, axis=1, keepdims=True)
              + jnp.sum(a3 * a3, axis=1, keepdims=True))
        m = s / (3.0 * c)
        var = ss / (3.0 * c) - m * m
        r = lax.rsqrt(var + _EPS)
        g1 = _sigmoid((a1 - m) * r * lag_ref[:, 0:c] + lab_ref[:, 0:c])
        g2 = _sigmoid((a2 - m) * r * lag_ref[:, c:2 * c] + lab_ref[:, c:2 * c])
        g3 = _sigmoid((a3 - m) * r * lag_ref[:, 2 * c:3 * c] + lab_ref[:, 2 * c:3 * c])
        m2 = (jnp.dot(v_i, wmu[0:c, :], preferred_element_type=jnp.float32)
              + jnp.dot(v_j, wmu[c:2 * c, :], preferred_element_type=jnp.float32)
              + jnp.dot(e, wmu[2 * c:3 * c, :], preferred_element_type=jnp.float32)
              + bmu)
        m3 = (jnp.dot(m2[:, 0:c] * g1, wml[0:c, :],
                      preferred_element_type=jnp.float32)
              + jnp.dot(m2[:, c:2 * c] * g2, wml[c:2 * c, :],
                        preferred_element_type=jnp.float32)
              + jnp.dot(m2[:, 2 * c:3 * c] * g3, wml[2 * c:3 * c, :],
                        preferred_element_type=jnp.float32)
              + bml)
        mm = jnp.mean(m3, axis=1, keepdims=True)
        mv = jnp.mean(m3 * m3, axis=1, keepdims=True) - mm * mm
        msg = (m3 - mm) * lax.rsqrt(mv + _EPS) * lmg_ref[...] + lmb_ref[...]
        o_ref[hh, :, :] = msg


def _make_gather(d, ep, b):
    per_tile = ep // (_NSC * _NTILE)
    nb = per_tile // b
    npair = nb // 2
    mesh = plsc.VectorSubcoreMesh(core_axis_name="c", subcore_axis_name="s")

    def body(table_hbm, idx_hbm, out_hbm, idx_v, r0, r1, g0, g1, o0, o1):
        cid = lax.axis_index("c")
        sid = lax.axis_index("s")
        wid = sid * _NSC + cid
        base = wid * per_tile
        pltpu.sync_copy(idx_hbm.at[pl.ds(base, per_tile)], idx_v)
        bufs = (r0, r1)
        gsem = (g0, g1)
        osem = (o0, o1)

        def start_g(j, p):
            pltpu.async_copy(table_hbm.at[idx_v.at[pl.ds(j * b, b)]],
                             bufs[p], gsem[p])

        def wait_g(j, p):
            pltpu.make_async_copy(table_hbm.at[idx_v.at[pl.ds(j * b, b)]],
                                  bufs[p], gsem[p]).wait()

        def start_o(j, p):
            pltpu.async_copy(bufs[p], out_hbm.at[pl.ds(base + j * b, b)],
                             osem[p])

        def wait_o(j, p):
            pltpu.make_async_copy(bufs[p],
                                  out_hbm.at[pl.ds(base + j * b, b)],
                                  osem[p]).wait()

        start_g(0, 0)
        start_g(1, 1)

        def pair(j2, carry):
            j = 2 * j2
            wait_g(j, 0)
            start_o(j, 0)
            wait_g(j + 1, 1)
            start_o(j + 1, 1)
            wait_o(j, 0)
            start_g(j + 2, 0)
            wait_o(j + 1, 1)
            start_g(j + 3, 1)
            return carry

        lax.fori_loop(0, npair - 1, pair, 0)
        j = nb - 2
        wait_g(j, 0)
        start_o(j, 0)
        wait_g(j + 1, 1)
        start_o(j + 1, 1)
        wait_o(j, 0)
        wait_o(j + 1, 1)

    def make(table, idx):
        return pl.kernel(
            body,
            out_type=jax.ShapeDtypeStruct((ep, d), jnp.int32),
            mesh=mesh,
            scratch_types=[
                pltpu.VMEM((per_tile,), jnp.int32),
                pltpu.VMEM((b, d), jnp.int32),
                pltpu.VMEM((b, d), jnp.int32),
                pltpu.SemaphoreType.DMA,
                pltpu.SemaphoreType.DMA,
                pltpu.SemaphoreType.DMA,
                pltpu.SemaphoreType.DMA,
            ],
        )(table, idx)

    return make


def _make_gather2(d1, d2, ep, b):
    per_tile = ep // (_NSC * _NTILE)
    nb = per_tile // b
    npair = nb // 2
    mesh = plsc.VectorSubcoreMesh(core_axis_name="c", subcore_axis_name="s")

    def body(tab1, idx1_h, tab2, idx2_h, out1, out2, ix1, ix2,
             a0, a1, b0, b1, ga0, ga1, gb0, gb1, oa0, oa1, ob0, ob1):
        cid = lax.axis_index("c")
        sid = lax.axis_index("s")
        wid = sid * _NSC + cid
        base = wid * per_tile
        pltpu.sync_copy(idx1_h.at[pl.ds(base, per_tile)], ix1)
        pltpu.sync_copy(idx2_h.at[pl.ds(base, per_tile)], ix2)
        abufs = (a0, a1)
        bbufs = (b0, b1)
        gas = (ga0, ga1)
        gbs = (gb0, gb1)
        oas = (oa0, oa1)
        obs = (ob0, ob1)

        def start_g(j, p):
            pltpu.async_copy(tab1.at[ix1.at[pl.ds(j * b, b)]],
                             abufs[p], gas[p])
            pltpu.async_copy(tab2.at[ix2.at[pl.ds(j * b, b)]],
                             bbufs[p], gbs[p])

        def wait_g(j, p):
            pltpu.make_async_copy(tab1.at[ix1.at[pl.ds(j * b, b)]],
                                  abufs[p], gas[p]).wait()
            pltpu.make_async_copy(tab2.at[ix2.at[pl.ds(j * b, b)]],
                                  bbufs[p], gbs[p]).wait()

        def start_o(j, p):
            pltpu.async_copy(abufs[p], out1.at[pl.ds(base + j * b, b)],
                             oas[p])
            pltpu.async_copy(bbufs[p], out2.at[pl.ds(base + j * b, b)],
                             obs[p])

        def wait_o(j, p):
            pltpu.make_async_copy(abufs[p],
                                  out1.at[pl.ds(base + j * b, b)],
                                  oas[p]).wait()
            pltpu.make_async_copy(bbufs[p],
                                  out2.at[pl.ds(base + j * b, b)],
                                  obs[p]).wait()

        start_g(0, 0)
        start_g(1, 1)

        def pair(j2, carry):
            j = 2 * j2
            wait_g(j, 0)
            start_o(j, 0)
            wait_g(j + 1, 1)
            start_o(j + 1, 1)
            wait_o(j, 0)
            start_g(j + 2, 0)
            wait_o(j + 1, 1)
            start_g(j + 3, 1)
            return carry

        lax.fori_loop(0, npair - 1, pair, 0)
        j = nb - 2
        wait_g(j, 0)
        start_o(j, 0)
        wait_g(j + 1, 1)
        start_o(j + 1, 1)
        wait_o(j, 0)
        wait_o(j + 1, 1)

    def make(tab1, idx1, tab2, idx2):
        return pl.kernel(
            body,
            out_type=(jax.ShapeDtypeStruct((ep, d1), jnp.int32),
                      jax.ShapeDtypeStruct((ep, d2), jnp.int32)),
            mesh=mesh,
            scratch_types=[
                pltpu.VMEM((per_tile,), jnp.int32),
                pltpu.VMEM((per_tile,), jnp.int32),
                pltpu.VMEM((b, d1), jnp.int32),
                pltpu.VMEM((b, d1), jnp.int32),
                pltpu.VMEM((b, d2), jnp.int32),
                pltpu.VMEM((b, d2), jnp.int32),
                pltpu.SemaphoreType.DMA,
                pltpu.SemaphoreType.DMA,
                pltpu.SemaphoreType.DMA,
                pltpu.SemaphoreType.DMA,
                pltpu.SemaphoreType.DMA,
                pltpu.SemaphoreType.DMA,
                pltpu.SemaphoreType.DMA,
                pltpu.SemaphoreType.DMA,
            ],
        )(tab1, idx1, tab2, idx2)

    return make


def _make_scatter(n, ep, h, c):
    ntab = ((n + _NTILE * 8) // (_NTILE * 8)) * (_NTILE * 8)
    br = 128
    per_tile_b = (ep // br) // _NTILE
    npair = per_tile_b // 2
    tab_slice = ntab // _NTILE
    rows_lo = (n // _NTILE) // 8 * 8
    rows_hi = n - rows_lo * (_NTILE - 1)
    vper = c // 16
    mesh = plsc.VectorSubcoreMesh(core_axis_name="c", subcore_axis_name="s")

    @functools.partial(
        pl.kernel,
        out_type=jax.ShapeDtypeStruct((h, n, c), jnp.float32),
        mesh=mesh,
        scratch_types=[
            pltpu.VMEM((128, c), jnp.float32),
            pltpu.VMEM((br, c), jnp.float32),
            pltpu.VMEM((br, c), jnp.float32),
            pltpu.VMEM((128,), jnp.int32),
            pltpu.VMEM((128,), jnp.int32),
            pltpu.VMEM_SHARED((ntab, c), jnp.float32),
            pltpu.SemaphoreType.DMA,
            pltpu.SemaphoreType.DMA,
        ],
    )
    def scat(msg_hbm, dstb_hbm, agg_hbm, zero_v, rows0, rows1,
             ia0, ia1, table, s0, s1):
        cid = lax.axis_index("c")
        sid = lax.axis_index("s")
        rows = (rows0, rows1)
        ia = (ia0, ia1)
        sem = (s0, s1)

        def zbody(i, carry):
            r = i // vper
            col = (i % vper) * 16
            zero_v[r, pl.ds(col, 16)] = jnp.zeros((16,), jnp.float32)
            return carry

        lax.fori_loop(0, 128 * vper, zbody, 0)
        tab_base = sid * tab_slice

        for ck in range(h // _NSC):
            chunk = cid * (h // _NSC) + ck
            off = 0
            left = tab_slice
            while left > 0:
                step = min(128, left)
                pltpu.sync_copy(zero_v.at[pl.ds(0, step)],
                                table.at[pl.ds(tab_base + off, step)])
                off += step
                left -= step
            plsc.subcore_barrier()

            def load(j, p):
                e0 = (sid * per_tile_b + j) * br
                pltpu.async_copy(dstb_hbm.at[pl.ds(e0, 128)], ia[p], sem[p])
                pltpu.async_copy(msg_hbm.at[chunk, pl.ds(e0, br)], rows[p],
                                 sem[p])

            def wait_load(j, p):
                e0 = (sid * per_tile_b + j) * br
                pltpu.make_async_copy(dstb_hbm.at[pl.ds(e0, 128)], ia[p],
                                      sem[p]).wait()
                pltpu.make_async_copy(msg_hbm.at[chunk, pl.ds(e0, br)],
                                      rows[p], sem[p]).wait()

            def add2(p):
                pltpu.sync_copy(rows[p], table.at[ia[p]], add=True)

            load(0, 0)

            def pair(j2, carry):
                j = 2 * j2
                load(j + 1, 1)
                wait_load(j, 0)
                add2(0)
                load(j + 2, 0)
                wait_load(j + 1, 1)
                add2(1)
                return carry

            lax.fori_loop(0, npair - 1, pair, 0)
            j = per_tile_b - 2
            load(j + 1, 1)
            wait_load(j, 0)
            add2(0)
            wait_load(j + 1, 1)
            add2(1)

            plsc.subcore_barrier()
            out_base = sid * rows_lo

            @pl.when(sid < _NTILE - 1)
            def _copy_lo():
                pltpu.sync_copy(table.at[pl.ds(out_base, rows_lo)],
                                agg_hbm.at[chunk, pl.ds(out_base, rows_lo)])

            @pl.when(sid == _NTILE - 1)
            def _copy_hi():
                base = rows_lo * (_NTILE - 1)
                pltpu.sync_copy(table.at[pl.ds(base, rows_hi)],
                                agg_hbm.at[chunk, pl.ds(base, rows_hi)])

            plsc.subcore_barrier()

    return scat


def _update_kernel(agg_ref, wcat_ref, bcat_ref, o_ref, stats_ref, *, h, c):
    i = pl.program_id(0)
    wcat = wcat_ref[...]
    o = bcat_ref[...]
    for hh in range(h):
        o = o + jnp.dot(agg_ref[hh, :, :], wcat[hh * c:(hh + 1) * c, :],
                        preferred_element_type=jnp.float32)
    o_ref[...] = o

    @pl.when(i == 0)
    def _init():
        stats_ref[...] = jnp.zeros_like(stats_ref)

    s = jnp.sum(o, axis=0, keepdims=True)
    ss = jnp.sum(o * o, axis=0, keepdims=True)
    stats_ref[0:1, :] = stats_ref[0:1, :] + s
    stats_ref[1:2, :] = stats_ref[1:2, :] + ss


def _pool_kernel(p_ref, stats_ref, g_ref, b_ref, batch_ref,
                 wfc_ref, bfc_ref, wout_ref, bout_ref, o_ref, *, n_nodes):
    p = p_ref[...]
    m = stats_ref[0:1, :] / n_nodes
    var = stats_ref[1:2, :] / n_nodes - m * m
    nf = _silu((p - m) * lax.rsqrt(var + _EPS) * g_ref[...] + b_ref[...])
    seg = batch_ref[...]
    gids = lax.broadcasted_iota(jnp.int32, (_G, 1), 0)
    onehot = (seg == gids).astype(jnp.float32)
    pooled = jnp.dot(onehot, nf, preferred_element_type=jnp.float32)
    counts = jnp.sum(onehot, axis=1, keepdims=True)
    pooled = pooled / jnp.maximum(counts, 1.0)
    feat = _silu(jnp.dot(pooled, wfc_ref[...],
                         preferred_element_type=jnp.float32) + bfc_ref[...])
    out = jnp.sum(feat * wout_ref[...], axis=1, keepdims=True) + bout_ref[...]
    o_ref[...] = out


def kernel(x, edge_attr, edge_index, batch, W_atom, b_atom, W_rbf1, b_rbf1,
           W_rbf2, b_rbf2, Wq, bq, Wk, bk, Wv, bv, We, Wcat, bcat, Wmu, bmu,
           Wml, bml, ln_msg_g, ln_msg_b, ln_a_g, ln_a_b, bn_g, bn_b,
           W_fc, b_fc, W_out, b_out):
    n, aif = x.shape
    e = edge_attr.shape[0]
    nfdim = W_atom.shape[1]
    efb = W_rbf1.shape[0]
    l_layers = Wq.shape[0]
    c = Wcat.shape[2]
    h = Wq.shape[2] // c
    hc = h * c

    ep = ((e + 4095) // 4096) * 4096
    pad = ep - e
    src = jnp.concatenate([edge_index[0], jnp.zeros((pad,), jnp.int32)])
    dst = jnp.concatenate([edge_index[1], jnp.zeros((pad,), jnp.int32)])
    dst_b = jnp.concatenate(
        [edge_index[1], jnp.full((pad,), n, jnp.int32)])
    edge_attr = jnp.concatenate(
        [edge_attr, jnp.zeros((pad, edge_attr.shape[1]), edge_attr.dtype)])

    f32 = jnp.float32
    row2 = lambda a: a.reshape(1, -1)

    nf0 = pl.pallas_call(
        _prologue_node_kernel,
        out_shape=jax.ShapeDtypeStruct((n, nfdim), f32),
    )(x, W_atom, row2(b_atom))

    eblk = _blk(ep, 5120)
    ef = pl.pallas_call(
        functools.partial(_prologue_edge_kernel, efb=efb),
        grid=(ep // eblk,),
        in_specs=[
            pl.BlockSpec((eblk, edge_attr.shape[1]), lambda i: (i, 0)),
            pl.BlockSpec((efb, nfdim), lambda i: (0, 0)),
            pl.BlockSpec((1, nfdim), lambda i: (0, 0)),
            pl.BlockSpec((nfdim, nfdim), lambda i: (0, 0)),
            pl.BlockSpec((1, nfdim), lambda i: (0, 0)),
        ],
        out_specs=pl.BlockSpec((eblk, nfdim), lambda i: (i, 0)),
        out_shape=jax.ShapeDtypeStruct((ep, nfdim), f32),
    )(edge_attr, W_rbf1, row2(b_rbf1), W_rbf2, row2(b_rbf2))

    nblk = _blk(n, 2000)
    ngrid = n // nblk

    prev = nf0
    stats = None
    for l in range(l_layers):
        i32 = jnp.int32
        nq = hc // c
        hp = nq // 2
        wspecs = [
            pl.BlockSpec((nfdim, hc), lambda i: (0, 0)),
            pl.BlockSpec((1, hc), lambda i: (0, 0)),
        ] * 3
        out_shapes = (jax.ShapeDtypeStruct((n, 3 * hp * 128), i32),
                      jax.ShapeDtypeStruct((n, nq * 128), i32))
        out_specs = (pl.BlockSpec((nblk, 3 * hp * 128), lambda i: (i, 0)),
                     pl.BlockSpec((nblk, nq * 128), lambda i: (i, 0)))
        wargs = (Wq[l], row2(bq[l]), Wk[l], row2(bk[l]), Wv[l], row2(bv[l]))
        if l == 0:
            qkv_t, kv_t = pl.pallas_call(
                functools.partial(_proj_kernel, hc=hc, c=c),
                grid=(ngrid,),
                in_specs=[pl.BlockSpec((nblk, nfdim), lambda i: (i, 0))] + wspecs,
                out_specs=out_specs,
                out_shape=out_shapes,
                    )(prev, *wargs)
        else:
            qkv_t, kv_t = pl.pallas_call(
                functools.partial(_proj_bn_kernel, hc=hc, c=c, n_nodes=n),
                grid=(ngrid,),
                in_specs=[
                    pl.BlockSpec((nblk, nfdim), lambda i: (i, 0)),
                    pl.BlockSpec((2, nfdim), lambda i: (0, 0)),
                    pl.BlockSpec((1, nfdim), lambda i: (0, 0)),
                    pl.BlockSpec((1, nfdim), lambda i: (0, 0)),
                ] + wspecs,
                out_specs=out_specs,
                out_shape=out_shapes,
                    )(prev, stats, row2(bn_g[l - 1]), row2(bn_b[l - 1]), *wargs)

        eblk2 = _blk(ep, 5120)
        e_t = pl.pallas_call(
            _eproj_kernel,
            grid=(ep // eblk2,),
            in_specs=[
                pl.BlockSpec((eblk2, nfdim), lambda i: (i, 0)),
                pl.BlockSpec((nfdim, hc), lambda i: (0, 0)),
            ],
            out_specs=pl.BlockSpec((eblk2, hc), lambda i: (i, 0)),
            out_shape=jax.ShapeDtypeStruct((ep, hc), f32),
            )(ef, We[l])

        g_qkv, g_kv = _make_gather2(3 * hp * 128, nq * 128, ep, 32)(
            qkv_t, dst, kv_t, src)

        eblk3 = _blk(ep, 1024)
        msg = pl.pallas_call(
            functools.partial(_edge_kernel, h=h, c=c),
            grid=(ep // eblk3,),
            in_specs=[
                pl.BlockSpec((eblk3, 3 * hp * 128), lambda i: (i, 0)),
                pl.BlockSpec((eblk3, nq * 128), lambda i: (i, 0)),
                pl.BlockSpec((eblk3, hc), lambda i: (i, 0)),
                pl.BlockSpec((3 * c, 3 * c), lambda i: (0, 0)),
                pl.BlockSpec((1, 3 * c), lambda i: (0, 0)),
                pl.BlockSpec((3 * c, c), lambda i: (0, 0)),
                pl.BlockSpec((1, c), lambda i: (0, 0)),
                pl.BlockSpec((1, 3 * c), lambda i: (0, 0)),
                pl.BlockSpec((1, 3 * c), lambda i: (0, 0)),
                pl.BlockSpec((1, c), lambda i: (0, 0)),
                pl.BlockSpec((1, c), lambda i: (0, 0)),
            ],
            out_specs=pl.BlockSpec((h, eblk3, c), lambda i: (0, i, 0)),
            out_shape=jax.ShapeDtypeStruct((h, ep, c), f32),
            )(g_qkv, g_kv, e_t, Wmu[l], row2(bmu[l]), Wml[l], row2(bml[l]),
          row2(ln_a_g[l]), row2(ln_a_b[l]), row2(ln_msg_g[l]),
          row2(ln_msg_b[l]))

        agg = _make_scatter(n, ep, h, c)(msg, dst_b)

        out_l, stats = pl.pallas_call(
            functools.partial(_update_kernel, h=h, c=c),
            grid=(ngrid,),
            in_specs=[
                pl.BlockSpec((h, nblk, c), lambda i: (0, i, 0)),
                pl.BlockSpec((hc, nfdim), lambda i: (0, 0)),
                pl.BlockSpec((1, nfdim), lambda i: (0, 0)),
            ],
            out_specs=(pl.BlockSpec((nblk, nfdim), lambda i: (i, 0)),
                       pl.BlockSpec((2, nfdim), lambda i: (0, 0))),
            out_shape=(jax.ShapeDtypeStruct((n, nfdim), f32),
                       jax.ShapeDtypeStruct((2, nfdim), f32)),
            )(agg, Wcat[l], row2(bcat[l]))
        prev = out_l

    out = pl.pallas_call(
        functools.partial(_pool_kernel, n_nodes=n),
        out_shape=jax.ShapeDtypeStruct((_G, 1), f32),
    )(prev, stats, row2(bn_g[l_layers - 1]), row2(bn_b[l_layers - 1]),
      batch.reshape(1, n), W_fc, row2(b_fc), W_out.reshape(1, -1),
      row2(b_out))
    return out.reshape(_G)
